# baseline (speedup 1.0000x reference)
import functools

import jax
import jax.numpy as jnp
import numpy as np
from jax import lax
from jax.experimental import pallas as pl
from jax.experimental.pallas import tpu as pltpu
from jax.experimental.pallas import tpu_sc as plsc

N = 10000
E = 160000
G = 8
NSPEC = 4
NWAVE = 16
PRMAXL = 3
PNORB = 9
MP_LOOP = 2
CUTOFF = 5.0
PN = 2.0
EPS = 1e-8
PIDX = (0, 1, 1, 1, 2, 2, 2, 2, 2)

EP = 163840
BLK = 2048
CH = 128

_NC = 2
_NS = 16
_NW = _NC * _NS
_PER_TILE = EP // _NW
_NCHUNK = _PER_TILE // CH
_NPAD = N
_NROWS = _NPAD // _NS
_PAD_NODE = N - 1
_ZROWS = 125


def _silu(x):
    return x * jax.nn.sigmoid(x)


def _sc_scatter_multi(vals_list, idx3d):
    nv = len(vals_list)
    Cs = [int(v.shape[1]) for v in vals_list]
    mesh = plsc.VectorSubcoreMesh(core_axis_name="c", subcore_axis_name="s")

    scratch = [pltpu.VMEM((_NCHUNK, CH), jnp.int32)]
    scratch += [pltpu.VMEM((2, CH, C), jnp.float32) for C in Cs]
    scratch += [pltpu.VMEM_SHARED((_NPAD, C), jnp.float32) for C in Cs]
    scratch += [pltpu.SemaphoreType.DMA] * (2 * nv)

    @functools.partial(
        pl.kernel, mesh=mesh,
        compiler_params=pltpu.CompilerParams(use_tc_tiling_on_sc=False),
        out_type=tuple(jax.ShapeDtypeStruct((_NC, _NPAD, C), jnp.float32)
                       for C in Cs),
        scratch_types=scratch,
    )
    def k(*refs):
        i = 0
        vals_hbm = refs[i:i + nv]; i += nv
        idx_hbm = refs[i]; i += 1
        zeros_hbm = refs[i:i + nv]; i += nv
        out_hbm = refs[i:i + nv]; i += nv
        idxv = refs[i]; i += 1
        bufs = refs[i:i + nv]; i += nv
        tabs = refs[i:i + nv]; i += nv
        sems = refs[i:i + 2 * nv]; i += 2 * nv

        c = lax.axis_index("c")
        s = lax.axis_index("s")
        wid = c * _NS + s
        base = wid * _PER_TILE
        for v in range(nv):
            for z in range(_NROWS // _ZROWS):
                pltpu.sync_copy(
                    zeros_hbm[v],
                    tabs[v].at[pl.ds(s * _NROWS + z * _ZROWS, _ZROWS), :])
        plsc.subcore_barrier()
        pltpu.sync_copy(idx_hbm.at[wid], idxv)

        def load(v, j, b):
            return pltpu.async_copy(
                vals_hbm[v].at[pl.ds(base + j * CH, CH), :],
                bufs[v].at[b], sems[2 * v + b])

        for v in range(nv):
            load(v, 0, 0)
            load(v, 1, 1)

        def step(j, b):
            for v in range(nv):
                pltpu.make_async_copy(
                    vals_hbm[v].at[pl.ds(base + j * CH, CH), :],
                    bufs[v].at[b], sems[2 * v + b]).wait()
                pltpu.sync_copy(bufs[v].at[b], tabs[v].at[idxv.at[j]],
                                add=True)

                @pl.when(j + 2 < _NCHUNK)
                def _():
                    load(v, j + 2, b)

        def outer(t, carry):
            step(2 * t, 0)
            step(2 * t + 1, 1)
            return carry

        lax.fori_loop(0, _NCHUNK // 2, outer, 0)
        plsc.subcore_barrier()
        for v in range(nv):
            pltpu.sync_copy(tabs[v].at[pl.ds(s * _NROWS, _NROWS), :],
                            out_hbm[v].at[c].at[pl.ds(s * _NROWS, _NROWS), :])

    zeros = [jnp.zeros((_ZROWS, C), jnp.float32) for C in Cs]
    parts = k(*vals_list, idx3d, *zeros)
    if not isinstance(parts, (tuple, list)):
        parts = (parts,)
    return [part[0, :N] + part[1, :N] for part in parts]


def _sc_gather2(table, idx3d_a, idx3d_b):
    C = table.shape[1]
    mesh = plsc.VectorSubcoreMesh(core_axis_name="c", subcore_axis_name="s")

    @functools.partial(
        pl.kernel, mesh=mesh,
        compiler_params=pltpu.CompilerParams(use_tc_tiling_on_sc=False),
        out_type=(jax.ShapeDtypeStruct((EP, C), jnp.float32),
                  jax.ShapeDtypeStruct((EP, C), jnp.float32)),
        scratch_types=[
            pltpu.VMEM((_NCHUNK, CH), jnp.int32),
            pltpu.VMEM((_NCHUNK, CH), jnp.int32),
            pltpu.VMEM((2, CH, C), jnp.float32),
            pltpu.VMEM((2, CH, C), jnp.float32),
            pltpu.SemaphoreType.DMA,
            pltpu.SemaphoreType.DMA,
            pltpu.SemaphoreType.DMA,
            pltpu.SemaphoreType.DMA,
        ],
    )
    def k(tab_hbm, ia_hbm, ib_hbm, outa_hbm, outb_hbm,
          idxa, idxb, bufa, bufb, sa0, sa1, sb0, sb1):
        c = lax.axis_index("c")
        s = lax.axis_index("s")
        wid = c * _NS + s
        base = wid * _PER_TILE
        sas = (sa0, sa1)
        sbs = (sb0, sb1)
        pltpu.sync_copy(ia_hbm.at[wid], idxa)
        pltpu.sync_copy(ib_hbm.at[wid], idxb)

        def issue(j, b):
            pltpu.async_copy(tab_hbm.at[idxa.at[j]], bufa.at[b], sas[b])
            pltpu.async_copy(tab_hbm.at[idxb.at[j]], bufb.at[b], sbs[b])

        issue(0, 0)
        issue(1, 1)

        def step(j, b):
            pltpu.make_async_copy(tab_hbm.at[idxa.at[j]], bufa.at[b],
                                  sas[b]).wait()
            pltpu.make_async_copy(tab_hbm.at[idxb.at[j]], bufb.at[b],
                                  sbs[b]).wait()
            pltpu.sync_copy(bufa.at[b],
                            outa_hbm.at[pl.ds(base + j * CH, CH), :])
            pltpu.sync_copy(bufb.at[b],
                            outb_hbm.at[pl.ds(base + j * CH, CH), :])

            @pl.when(j + 2 < _NCHUNK)
            def _():
                issue(j + 2, b)

        def outer(t, carry):
            step(2 * t, 0)
            step(2 * t + 1, 1)
            return carry

        lax.fori_loop(0, _NCHUNK // 2, outer, 0)

    return k(table, idx3d_a, idx3d_b)


def _edge1_body(gs_ref, gd_ref, sh_ref, cellm_ref, embt_ref, ieadt_ref,
                rdW1_ref, rdW2_ref, e2W1_ref, e2W2_ref,
                sph_ref, ead_ref, wdc_ref, worbA_ref, worbB_ref, rad_ref):
    gs = gs_ref[...]
    gd = gd_ref[...]
    sh = sh_ref[...]
    cellm = cellm_ref[...]
    nedge = gs.shape[0]

    cidx = gs[:, 5:6]
    cm = None
    for g in range(G):
        term = (cidx == float(g)).astype(jnp.float32) * cellm[g:g + 1, :]
        cm = term if cm is None else cm + term
    sv = []
    for kk in range(3):
        sv.append(sh[:, 0:1] * cm[:, kk:kk + 1]
                  + sh[:, 1:2] * cm[:, 3 + kk:4 + kk]
                  + sh[:, 2:3] * cm[:, 6 + kk:7 + kk])

    dx = gd[:, 0:1] - gs[:, 0:1] + sv[0]
    dy = gd[:, 1:2] - gs[:, 1:2] + sv[1]
    dz = gd[:, 2:3] - gs[:, 2:3] + sv[2]
    distsq = dx * dx + dy * dy + dz * dz
    nf = (distsq > EPS).astype(jnp.float32)
    dist = jnp.sqrt(distsq + EPS)
    inv = 1.0 / dist
    ux = dx * inv
    uy = dy * inv
    uz = dz * inv
    s = [jnp.ones_like(ux), ux, uy, uz, ux * uy, uy * uz,
         3.0 * uz * uz - 1.0, uz * ux, ux * ux - uy * uy]
    n0 = jnp.ones_like(ux) + EPS
    n1 = ux * ux + uy * uy + uz * uz + EPS
    n2 = (s[4] * s[4] + s[5] * s[5] + s[6] * s[6] + s[7] * s[7]
          + s[8] * s[8] + EPS)
    f = [lax.rsqrt(n0), jnp.sqrt(3.0) * lax.rsqrt(n1),
         jnp.sqrt(5.0) * lax.rsqrt(n2)]
    sph = [s[j] * f[PIDX[j]] for j in range(PNORB)]
    sph_ref[...] = jnp.concatenate(
        sph + [jnp.zeros((nedge, NWAVE - PNORB), jnp.float32)], axis=1)

    nd = dist * (1.0 / CUTOFF)
    poly = 1.0 - nd * nd * ((PN + 1.0) * (PN + 2.0) / 2.0
                            - PN * (PN + 2.0) * nd
                            + PN * (PN + 1.0) / 2.0 * nd * nd)
    cut = poly * poly * nf

    pidx = gs[:, 4:5] * float(NSPEC) + gd[:, 4:5]
    embt = embt_ref[...]
    ieadt = ieadt_ref[...]
    embc = None
    iead = None
    for q in range(NSPEC * NSPEC):
        oh = (pidx == float(q)).astype(jnp.float32)
        te = oh * embt[q:q + 1, :]
        ti = oh * ieadt[q:q + 1, :]
        embc = te if embc is None else embc + te
        iead = ti if iead is None else iead + ti

    smooth = iead * cut
    rf = jnp.sinc(nd * embc) * cut
    radial_func = jnp.concatenate([smooth[:, NWAVE:], rf], axis=1)
    h = _silu(jnp.dot(radial_func, rdW1_ref[...],
                      preferred_element_type=jnp.float32))
    wr = jnp.dot(h, rdW2_ref[...], preferred_element_type=jnp.float32)
    ead = jnp.concatenate([smooth[:, :NWAVE], wr[:, 4 * NWAVE:]], axis=1)
    ead_ref[...] = ead
    wdc_ref[...] = jnp.concatenate(
        [wr[:, 3 * NWAVE:4 * NWAVE], cut,
         jnp.zeros((nedge, NWAVE - 1), jnp.float32)], axis=1)
    worbA_ref[...] = jnp.concatenate(
        [wr[:, PIDX[j] * NWAVE:(PIDX[j] + 1) * NWAVE] * sph[j]
         for j in range(5)], axis=1)
    worbB_ref[...] = jnp.concatenate(
        [wr[:, PIDX[j] * NWAVE:(PIDX[j] + 1) * NWAVE] * sph[j]
         for j in range(5, PNORB)], axis=1)
    h2 = _silu(jnp.dot(ead, e2W1_ref[...],
                       preferred_element_type=jnp.float32))
    rad_ref[...] = jnp.dot(h2, e2W2_ref[...],
                           preferred_element_type=jnp.float32)


def _edge_phase1(gs, gd, shT, cellm, embt, ieadt, rdW1, rdW2, e2W1, e2W2):
    def eb(c):
        return pl.BlockSpec((BLK, c), lambda i: (i, 0))

    def wb(shape):
        return pl.BlockSpec(shape, lambda i: (0, 0))

    outs = (
        jax.ShapeDtypeStruct((EP, NWAVE), jnp.float32),
        jax.ShapeDtypeStruct((EP, 2 * NWAVE), jnp.float32),
        jax.ShapeDtypeStruct((EP, 2 * NWAVE), jnp.float32),
        jax.ShapeDtypeStruct((EP, 5 * NWAVE), jnp.float32),
        jax.ShapeDtypeStruct((EP, 4 * NWAVE), jnp.float32),
        jax.ShapeDtypeStruct((EP, 3 * PRMAXL * NWAVE), jnp.float32),
    )
    return pl.pallas_call(
        _edge1_body,
        grid=(EP // BLK,),
        in_specs=[eb(16), eb(16), eb(4), wb(cellm.shape), wb(embt.shape),
                  wb(ieadt.shape), wb(rdW1.shape), wb(rdW2.shape),
                  wb(e2W1.shape), wb(e2W2.shape)],
        out_specs=(eb(NWAVE), eb(2 * NWAVE), eb(2 * NWAVE),
                   eb(5 * NWAVE), eb(4 * NWAVE), eb(3 * PRMAXL * NWAVE)),
        out_shape=outs,
    )(gs, gd, shT, cellm, embt, ieadt, rdW1, rdW2, e2W1, e2W2)


def _edge2_body(has_ead_out, ead_parts, refs):
    i = 0
    eads = []
    for _ in range(ead_parts):
        eads.append(refs[i][...])
        i += 1
    sph = refs[i][...]; i += 1
    rad = refs[i][...]; i += 1
    nc0 = refs[i][...]; i += 1
    nc1 = refs[i][...]; i += 1
    mpW1 = refs[i][...]; i += 1
    mpW2 = refs[i][...]; i += 1
    if has_ead_out:
        eW1 = refs[i][...]; i += 1
        eW2 = refs[i][...]; i += 1
    ne_ref = refs[i]; i += 1
    orbA_ref = refs[i]; i += 1
    orbB_ref = refs[i]; i += 1
    nworbA_ref = refs[i]; i += 1
    nworbB_ref = refs[i]; i += 1
    if has_ead_out:
        radnew_ref = refs[i]; i += 1

    def rrow(r, ppp):
        col = (r * PRMAXL + ppp) * NWAVE
        return rad[:, col:col + NWAVE]

    ne = None
    orb_blocks = []
    for j in range(PNORB):
        pj = PIDX[j]
        sl = slice(j * NWAVE, (j + 1) * NWAVE)
        ao = rrow(0, pj) * nc0[:, sl] + rrow(1, pj) * nc1[:, sl]
        contrib = sph[:, j:j + 1] * ao
        ne = contrib if ne is None else ne + contrib
        orb_blocks.append(rrow(2, pj) * sph[:, j:j + 1])
    ne = ne * (1.0 / np.sqrt(2.0))
    ne_ref[...] = ne
    orbA_ref[...] = jnp.concatenate(orb_blocks[:5], axis=1)
    orbB_ref[...] = jnp.concatenate(orb_blocks[5:], axis=1)

    ead_cat = jnp.concatenate(eads + [ne], axis=1)
    h = _silu(jnp.dot(ead_cat, mpW1, preferred_element_type=jnp.float32))
    wr = jnp.dot(h, mpW2, preferred_element_type=jnp.float32)
    nwb = [wr[:, PIDX[j] * NWAVE:(PIDX[j] + 1) * NWAVE] * sph[:, j:j + 1]
           for j in range(PNORB)]
    nworbA_ref[...] = jnp.concatenate(nwb[:5], axis=1)
    nworbB_ref[...] = jnp.concatenate(nwb[5:], axis=1)
    if has_ead_out:
        h2 = _silu(jnp.dot(ead_cat, eW1, preferred_element_type=jnp.float32))
        radnew_ref[...] = jnp.dot(h2, eW2,
                                  preferred_element_type=jnp.float32)


def _edge_phase2(ead_list, sph, rad, nc0, nc1, mpW1, mpW2, eadW=None):
    def eb(c):
        return pl.BlockSpec((BLK, c), lambda i: (i, 0))

    def wb(shape):
        return pl.BlockSpec(shape, lambda i: (0, 0))

    has_ead_out = eadW is not None
    n_ead = len(ead_list)
    in_specs = ([eb(e.shape[1]) for e in ead_list]
                + [eb(NWAVE), eb(3 * PRMAXL * NWAVE),
                   eb(PNORB * NWAVE), eb(PNORB * NWAVE),
                   wb(mpW1.shape), wb(mpW2.shape)])
    args = list(ead_list) + [sph, rad, nc0, nc1, mpW1, mpW2]
    if has_ead_out:
        in_specs += [wb(eadW[0].shape), wb(eadW[1].shape)]
        args += [eadW[0], eadW[1]]
    outs = [jax.ShapeDtypeStruct((EP, NWAVE), jnp.float32),
            jax.ShapeDtypeStruct((EP, 5 * NWAVE), jnp.float32),
            jax.ShapeDtypeStruct((EP, 4 * NWAVE), jnp.float32),
            jax.ShapeDtypeStruct((EP, 5 * NWAVE), jnp.float32),
            jax.ShapeDtypeStruct((EP, 4 * NWAVE), jnp.float32)]
    out_specs = [eb(NWAVE), eb(5 * NWAVE), eb(4 * NWAVE),
                 eb(5 * NWAVE), eb(4 * NWAVE)]
    if has_ead_out:
        outs.append(jax.ShapeDtypeStruct((EP, 3 * PRMAXL * NWAVE),
                                         jnp.float32))
        out_specs.append(eb(3 * PRMAXL * NWAVE))

    def body(*refs):
        _edge2_body(has_ead_out, n_ead, refs)

    return pl.pallas_call(
        body,
        grid=(EP // BLK,),
        in_specs=in_specs,
        out_specs=tuple(out_specs),
        out_shape=tuple(outs),
    )(*args)


def kernel(cart, cell, disp_cell, neighlist, celllist, shiftimage,
           center_factor, species, params):
    p = params
    f32 = jnp.float32
    com_spec = jnp.array([[float(i), float(j)] for i in range(NSPEC)
                          for j in range(NSPEC)], dtype=f32)

    symm_cell = (disp_cell + jnp.transpose(disp_cell, (0, 2, 1))) / 2.0
    cell = cell + jnp.einsum('ijk,ikm->ijm', cell, symm_cell)
    symm_cell_n = symm_cell[celllist]
    cart = cart + jnp.einsum('ij,ijk->ik', cart, symm_cell_n)
    cellm = jnp.concatenate(
        [cell.reshape(G, 9), jnp.zeros((G, 7), f32)], axis=1)

    pad_idx = jnp.full((EP - E,), _PAD_NODE, jnp.int32)
    idx0 = jnp.concatenate([neighlist[0], pad_idx]).reshape(_NW, _NCHUNK, CH)
    idx1 = jnp.concatenate([neighlist[1], pad_idx]).reshape(_NW, _NCHUNK, CH)
    spec_idx = species

    node_tab = jnp.concatenate(
        [cart, jnp.zeros((N, 1), f32), spec_idx[:, None].astype(f32),
         celllist[:, None].astype(f32), jnp.zeros((N, 10), f32)], axis=1)
    node_tab = jnp.concatenate(
        [node_tab, jnp.zeros((_NPAD - N, 16), f32)], axis=0)
    gs, gd = _sc_gather2(node_tab, idx0, idx1)

    shT = jnp.concatenate(
        [jnp.concatenate([shiftimage.T, jnp.zeros((E, 1), f32)], axis=1),
         jnp.zeros((EP - E, 4), f32)], axis=0)

    pair_spec = _silu(com_spec @ p['ncW1'] + p['ncB1']) @ p['ncW2'] + p['ncB2']
    embt = (_silu(pair_spec @ p['nnW1'] + p['nnB1']) @ p['nnW2']
            + p['nnB2'])
    ieadt = _silu(pair_spec @ p['rwW1']) @ p['rwW2']

    sph, ead0, wdc, worbA, worbB, rad = _edge_phase1(
        gs, gd, shT, cellm, embt, ieadt, p['rdW1'], p['rdW2'],
        p['ead2W1'], p['ead2W2'])

    wdc_n = _sc_scatter_multi([wdc], idx0)[0]
    density = wdc_n[:, :NWAVE]
    ave = wdc_n[:, NWAVE:NWAVE + 1] + EPS
    corb = jnp.concatenate(
        [_sc_scatter_multi([worbA], idx0)[0],
         _sc_scatter_multi([worbB], idx0)[0]],
        axis=1).reshape(N, PNORB, NWAVE)
    spec_co = (p['spec_coeff'] / np.sqrt(float(NWAVE)))[spec_idx]
    corb = jnp.einsum('ikm,ijk->ijm', spec_co, corb / ave[:, None])

    ead_list = [ead0]
    mpW = [(p['mp0W1'], p['mp0W2']), (p['mp1W1'], p['mp1W2'])]
    for it in range(MP_LOOP):
        norm_corb = corb * (1.0 / np.sqrt(float(PRMAXL)))
        ncf = norm_corb.reshape(N, PNORB * NWAVE)
        ncf_pad = jnp.concatenate(
            [ncf, jnp.zeros((_NPAD - N, PNORB * NWAVE), f32)], axis=0)
        nc0, nc1 = _sc_gather2(ncf_pad, idx0, idx1)
        eadW = ((p['ead0W1'], p['ead0W2']) if it < MP_LOOP - 1 else None)
        res = _edge_phase2(ead_list, sph, rad, nc0, nc1,
                           mpW[it][0], mpW[it][1], eadW)
        ne, orbA, orbB, nworbA, nworbB = res[0], res[1], res[2], res[3], res[4]
        if eadW is not None:
            rad = res[5]
        ead_list = ead_list + [ne]
        sum_orb = jnp.concatenate(
            [_sc_scatter_multi([orbA], idx0)[0],
             _sc_scatter_multi([orbB], idx0)[0]],
            axis=1).reshape(N, PNORB, NWAVE)
        density1 = jnp.sum(sum_orb * norm_corb, axis=1)
        density = jnp.concatenate([density, density1], axis=1)
        sum_new = jnp.concatenate(
            [_sc_scatter_multi([nworbA], idx0)[0],
             _sc_scatter_multi([nworbB], idx0)[0]],
            axis=1).reshape(N, PNORB, NWAVE)
        cc = p['contract_coeff'][it][spec_idx]
        corb = (sum_new / ave[:, None]) * 0.1 + corb * 0.2
        lmod = jnp.transpose(p['l_coeff'][it][:, spec_idx], (1, 0, 2))
        corb = corb * lmod

    atom_out = (_silu(density @ p['outW1'] + p['outB1']) @ p['outW2']
                + p['outB2'])[:, 0]
    sc = p['scale'].reshape(NSPEC, 2)[spec_idx]
    atom_energy = (atom_out * sc[:, 0] + sc[:, 1]) * center_factor
    energy = jax.ops.segment_sum(atom_energy, celllist, num_segments=G,
                                 indices_are_sorted=True)
    return energy


# speedup vs baseline: 1.5645x; 1.5645x over previous
import functools

import jax
import jax.numpy as jnp
import numpy as np
from jax import lax
from jax.experimental import pallas as pl
from jax.experimental.pallas import tpu as pltpu
from jax.experimental.pallas import tpu_sc as plsc

N = 10000
E = 160000
G = 8
NSPEC = 4
NWAVE = 16
PRMAXL = 3
PNORB = 9
MP_LOOP = 2
CUTOFF = 5.0
PN = 2.0
EPS = 1e-8
PIDX = (0, 1, 1, 1, 2, 2, 2, 2, 2)

EP = 163840
BLK = 2048
CH = 128

_NC = 2
_NS = 16
_NW = _NC * _NS
_PER_TILE = EP // _NW
_NCHUNK = _PER_TILE // CH
_NPAD = N
_NROWS = _NPAD // _NS
_PAD_NODE = N - 1
_ZROWS = 125


def _silu(x):
    return x * jax.nn.sigmoid(x)


def _sc_scatter_multi(vals_list, idx3d):
    nv = len(vals_list)
    Cs = [int(v.shape[1]) for v in vals_list]
    mesh = plsc.VectorSubcoreMesh(core_axis_name="c", subcore_axis_name="s")

    scratch = [pltpu.VMEM((_NCHUNK, CH), jnp.int32)]
    scratch += [pltpu.VMEM((2, CH, C), jnp.float32) for C in Cs]
    scratch += [pltpu.VMEM_SHARED((_NPAD, C), jnp.float32) for C in Cs]
    scratch += [pltpu.SemaphoreType.DMA] * (2 * nv)

    @functools.partial(
        pl.kernel, mesh=mesh,
        compiler_params=pltpu.CompilerParams(use_tc_tiling_on_sc=False),
        out_type=tuple(jax.ShapeDtypeStruct((_NC, _NPAD, C), jnp.float32)
                       for C in Cs),
        scratch_types=scratch,
    )
    def k(*refs):
        i = 0
        vals_hbm = refs[i:i + nv]; i += nv
        idx_hbm = refs[i]; i += 1
        zeros_hbm = refs[i:i + nv]; i += nv
        out_hbm = refs[i:i + nv]; i += nv
        idxv = refs[i]; i += 1
        bufs = refs[i:i + nv]; i += nv
        tabs = refs[i:i + nv]; i += nv
        sems = refs[i:i + 2 * nv]; i += 2 * nv

        c = lax.axis_index("c")
        s = lax.axis_index("s")
        wid = c * _NS + s
        base = wid * _PER_TILE
        for v in range(nv):
            for z in range(_NROWS // _ZROWS):
                pltpu.sync_copy(
                    zeros_hbm[v],
                    tabs[v].at[pl.ds(s * _NROWS + z * _ZROWS, _ZROWS), :])
        plsc.subcore_barrier()
        pltpu.sync_copy(idx_hbm.at[wid], idxv)

        def load(v, j, b):
            return pltpu.async_copy(
                vals_hbm[v].at[pl.ds(base + j * CH, CH), :],
                bufs[v].at[b], sems[2 * v + b])

        for v in range(nv):
            load(v, 0, 0)
            load(v, 1, 1)

        def step(j, b):
            for v in range(nv):
                pltpu.make_async_copy(
                    vals_hbm[v].at[pl.ds(base + j * CH, CH), :],
                    bufs[v].at[b], sems[2 * v + b]).wait()
                pltpu.sync_copy(bufs[v].at[b], tabs[v].at[idxv.at[j]],
                                add=True)

                @pl.when(j + 2 < _NCHUNK)
                def _():
                    load(v, j + 2, b)

        def outer(t, carry):
            step(2 * t, 0)
            step(2 * t + 1, 1)
            return carry

        lax.fori_loop(0, _NCHUNK // 2, outer, 0)
        plsc.subcore_barrier()
        for v in range(nv):
            pltpu.sync_copy(tabs[v].at[pl.ds(s * _NROWS, _NROWS), :],
                            out_hbm[v].at[c].at[pl.ds(s * _NROWS, _NROWS), :])

    zeros = [jnp.zeros((_ZROWS, C), jnp.float32) for C in Cs]
    parts = k(*vals_list, idx3d, *zeros)
    if not isinstance(parts, (tuple, list)):
        parts = (parts,)
    return [part[0, :N] + part[1, :N] for part in parts]


def _sc_gather2(table, idx3d_a, idx3d_b):
    C = table.shape[1]
    mesh = plsc.VectorSubcoreMesh(core_axis_name="c", subcore_axis_name="s")

    @functools.partial(
        pl.kernel, mesh=mesh,
        compiler_params=pltpu.CompilerParams(use_tc_tiling_on_sc=False),
        out_type=(jax.ShapeDtypeStruct((EP, C), jnp.float32),
                  jax.ShapeDtypeStruct((EP, C), jnp.float32)),
        scratch_types=[
            pltpu.VMEM((_NCHUNK, CH), jnp.int32),
            pltpu.VMEM((_NCHUNK, CH), jnp.int32),
            pltpu.VMEM((2, CH, C), jnp.float32),
            pltpu.VMEM((2, CH, C), jnp.float32),
            pltpu.SemaphoreType.DMA,
            pltpu.SemaphoreType.DMA,
            pltpu.SemaphoreType.DMA,
            pltpu.SemaphoreType.DMA,
        ],
    )
    def k(tab_hbm, ia_hbm, ib_hbm, outa_hbm, outb_hbm,
          idxa, idxb, bufa, bufb, sa0, sa1, sb0, sb1):
        c = lax.axis_index("c")
        s = lax.axis_index("s")
        wid = c * _NS + s
        base = wid * _PER_TILE
        sas = (sa0, sa1)
        sbs = (sb0, sb1)
        pltpu.sync_copy(ia_hbm.at[wid], idxa)
        pltpu.sync_copy(ib_hbm.at[wid], idxb)

        def issue(j, b):
            pltpu.async_copy(tab_hbm.at[idxa.at[j]], bufa.at[b], sas[b])
            pltpu.async_copy(tab_hbm.at[idxb.at[j]], bufb.at[b], sbs[b])

        issue(0, 0)
        issue(1, 1)

        def step(j, b):
            pltpu.make_async_copy(tab_hbm.at[idxa.at[j]], bufa.at[b],
                                  sas[b]).wait()
            pltpu.make_async_copy(tab_hbm.at[idxb.at[j]], bufb.at[b],
                                  sbs[b]).wait()
            pltpu.sync_copy(bufa.at[b],
                            outa_hbm.at[pl.ds(base + j * CH, CH), :])
            pltpu.sync_copy(bufb.at[b],
                            outb_hbm.at[pl.ds(base + j * CH, CH), :])

            @pl.when(j + 2 < _NCHUNK)
            def _():
                issue(j + 2, b)

        def outer(t, carry):
            step(2 * t, 0)
            step(2 * t + 1, 1)
            return carry

        lax.fori_loop(0, _NCHUNK // 2, outer, 0)

    return k(table, idx3d_a, idx3d_b)


def _edge1_body(gs_ref, gd_ref, sh_ref, cellm_ref, embt_ref, ieadt_ref,
                rdW1_ref, rdW2a_ref, rdW2bA_ref, rdW2bB_ref,
                e2W1_ref, e2W2e_ref, psA_ref, psB_ref,
                geo_ref, ead_ref, wdc_ref, worbA_ref, worbB_ref, rad_ref):
    gst = jnp.transpose(gs_ref[...])
    gdt = jnp.transpose(gd_ref[...])
    sht = sh_ref[...]
    cellm = cellm_ref[...]
    nedge = gs_ref.shape[0]

    xs, ys, zs = gst[0:1], gst[1:2], gst[2:3]
    xd, yd, zd = gdt[0:1], gdt[1:2], gdt[2:3]
    spec_s, cidx = gst[4:5], gst[5:6]
    spec_d = gdt[4:5]

    oh = [(cidx == float(g)).astype(jnp.float32) for g in range(G)]
    cmv = []
    for q in range(9):
        acc = None
        for g in range(G):
            t = oh[g] * cellm[g:g + 1, q:q + 1]
            acc = t if acc is None else acc + t
        cmv.append(acc)
    sv = [sht[0:1] * cmv[0 + kk] + sht[1:2] * cmv[3 + kk]
          + sht[2:3] * cmv[6 + kk] for kk in range(3)]

    dx = xd - xs + sv[0]
    dy = yd - ys + sv[1]
    dz = zd - zs + sv[2]
    distsq = dx * dx + dy * dy + dz * dz
    nf = (distsq > EPS).astype(jnp.float32)
    dist = jnp.sqrt(distsq + EPS)
    inv = 1.0 / dist
    ux, uy, uz = dx * inv, dy * inv, dz * inv
    s = [jnp.ones_like(ux), ux, uy, uz, ux * uy, uy * uz,
         3.0 * uz * uz - 1.0, uz * ux, ux * ux - uy * uy]
    n1 = ux * ux + uy * uy + uz * uz + EPS
    n2 = (s[4] * s[4] + s[5] * s[5] + s[6] * s[6] + s[7] * s[7]
          + s[8] * s[8] + EPS)
    f = [lax.rsqrt(jnp.ones_like(ux) + EPS),
         jnp.sqrt(3.0) * lax.rsqrt(n1),
         jnp.sqrt(5.0) * lax.rsqrt(n2)]
    sph = [s[j] * f[PIDX[j]] for j in range(PNORB)]

    nd = dist * (1.0 / CUTOFF)
    poly = 1.0 - nd * nd * ((PN + 1.0) * (PN + 2.0) / 2.0
                            - PN * (PN + 2.0) * nd
                            + PN * (PN + 1.0) / 2.0 * nd * nd)
    cut = poly * poly * nf
    pi = spec_s * float(NSPEC) + spec_d

    geo_rows = jnp.concatenate(
        sph + [nd, cut, pi, jnp.zeros((4, nedge), jnp.float32)], axis=0)
    geo = jnp.transpose(geo_rows)
    geo_ref[...] = geo

    nd_c = geo[:, 9:10]
    cut_c = geo[:, 10:11]
    pi_c = geo[:, 11:12]
    lane = lax.broadcasted_iota(
        jnp.int32, (1, NSPEC * NSPEC), 1).astype(jnp.float32)
    ohp = (pi_c == lane).astype(jnp.float32)
    embc = jnp.dot(ohp, embt_ref[...], preferred_element_type=jnp.float32)
    iead = jnp.dot(ohp, ieadt_ref[...], preferred_element_type=jnp.float32)

    smooth = iead * cut_c
    rf = jnp.sinc(nd_c * embc) * cut_c
    radial_func = jnp.concatenate([smooth[:, NWAVE:], rf], axis=1)
    h = _silu(jnp.dot(radial_func, rdW1_ref[...],
                      preferred_element_type=jnp.float32))
    wr_a = jnp.dot(h, rdW2a_ref[...], preferred_element_type=jnp.float32)
    ead = jnp.concatenate([smooth[:, :NWAVE], wr_a[:, NWAVE:]], axis=1)
    ead_ref[...] = ead
    wdc_ref[...] = jnp.concatenate(
        [wr_a[:, :NWAVE], cut_c,
         jnp.zeros((nedge, NWAVE - 1), jnp.float32)], axis=1)

    sphxA = jnp.dot(geo, psA_ref[...], preferred_element_type=jnp.float32)
    sphxB = jnp.dot(geo, psB_ref[...], preferred_element_type=jnp.float32)
    worbA_ref[...] = jnp.dot(h, rdW2bA_ref[...],
                             preferred_element_type=jnp.float32) * sphxA
    worbB_ref[...] = jnp.dot(h, rdW2bB_ref[...],
                             preferred_element_type=jnp.float32) * sphxB

    h2 = _silu(jnp.dot(ead, e2W1_ref[...],
                       preferred_element_type=jnp.float32))
    rad_ref[...] = jnp.dot(h2, e2W2e_ref[...],
                           preferred_element_type=jnp.float32)


def _edge_phase1(gs, gd, shp, cellm, embt, ieadt, rdW1, rdW2a, rdW2bA,
                 rdW2bB, e2W1, e2W2e, psA, psB):
    def eb(c):
        return pl.BlockSpec((BLK, c), lambda i: (i, 0))

    def wb(shape):
        return pl.BlockSpec(shape, lambda i: tuple(0 for _ in shape))

    outs = (
        jax.ShapeDtypeStruct((EP, NWAVE), jnp.float32),
        jax.ShapeDtypeStruct((EP, 2 * NWAVE), jnp.float32),
        jax.ShapeDtypeStruct((EP, 2 * NWAVE), jnp.float32),
        jax.ShapeDtypeStruct((EP, 5 * NWAVE), jnp.float32),
        jax.ShapeDtypeStruct((EP, 4 * NWAVE), jnp.float32),
        jax.ShapeDtypeStruct((EP, 27 * NWAVE), jnp.float32),
    )
    return pl.pallas_call(
        _edge1_body,
        grid=(EP // BLK,),
        in_specs=[eb(16), eb(16),
                  pl.BlockSpec((8, BLK), lambda i: (0, i)),
                  wb(cellm.shape), wb(embt.shape), wb(ieadt.shape),
                  wb(rdW1.shape), wb(rdW2a.shape), wb(rdW2bA.shape),
                  wb(rdW2bB.shape), wb(e2W1.shape), wb(e2W2e.shape),
                  wb(psA.shape), wb(psB.shape)],
        out_specs=(eb(NWAVE), eb(2 * NWAVE), eb(2 * NWAVE),
                   eb(5 * NWAVE), eb(4 * NWAVE), eb(27 * NWAVE)),
        out_shape=outs,
    )(gs, gd, shp, cellm, embt, ieadt, rdW1, rdW2a, rdW2bA, rdW2bB,
      e2W1, e2W2e, psA, psB)


def _edge2_body(has_ead_out, ead_parts, refs):
    i = 0
    eads = []
    for _ in range(ead_parts):
        eads.append(refs[i][...])
        i += 1
    geo = refs[i][...]; i += 1
    rad = refs[i][...]; i += 1
    nc0 = refs[i][...]; i += 1
    nc1 = refs[i][...]; i += 1
    mpW1 = refs[i][...]; i += 1
    mpW2e = refs[i][...]; i += 1
    psA = refs[i][...]; i += 1
    psB = refs[i][...]; i += 1
    qmat = refs[i][...]; i += 1
    if has_ead_out:
        eW1 = refs[i][...]; i += 1
        eW2e = refs[i][...]; i += 1
    ne_ref = refs[i]; i += 1
    orbA_ref = refs[i]; i += 1
    orbB_ref = refs[i]; i += 1
    nworbA_ref = refs[i]; i += 1
    nworbB_ref = refs[i]; i += 1
    if has_ead_out:
        radnew_ref = refs[i]; i += 1

    W = PNORB * NWAVE
    sphxA = jnp.dot(geo, psA, preferred_element_type=jnp.float32)
    sphxB = jnp.dot(geo, psB, preferred_element_type=jnp.float32)
    r0A, r0B = rad[:, 0:80], rad[:, 80:W]
    r1A, r1B = rad[:, W:W + 80], rad[:, W + 80:2 * W]
    r2A, r2B = rad[:, 2 * W:2 * W + 80], rad[:, 2 * W + 80:3 * W]
    aoA = (r0A * nc0[:, :80] + r1A * nc1[:, :80]) * sphxA
    aoB = (r0B * nc0[:, 80:] + r1B * nc1[:, 80:]) * sphxB
    ne = (jnp.dot(aoA, qmat[:80], preferred_element_type=jnp.float32)
          + jnp.dot(aoB, qmat[80:], preferred_element_type=jnp.float32))
    ne_ref[...] = ne
    orbA_ref[...] = r2A * sphxA
    orbB_ref[...] = r2B * sphxB

    ead_cat = jnp.concatenate(eads + [ne], axis=1)
    h = _silu(jnp.dot(ead_cat, mpW1, preferred_element_type=jnp.float32))
    nwA = jnp.dot(h, mpW2e[:, :80], preferred_element_type=jnp.float32)
    nwB = jnp.dot(h, mpW2e[:, 80:], preferred_element_type=jnp.float32)
    nworbA_ref[...] = nwA * sphxA
    nworbB_ref[...] = nwB * sphxB
    if has_ead_out:
        h2 = _silu(jnp.dot(ead_cat, eW1, preferred_element_type=jnp.float32))
        radnew_ref[...] = jnp.dot(h2, eW2e,
                                  preferred_element_type=jnp.float32)


def _edge_phase2(ead_list, geo, rad, nc0, nc1, mpW1, mpW2e, psA, psB, qmat,
                 eadW=None):
    def eb(c):
        return pl.BlockSpec((BLK, c), lambda i: (i, 0))

    def wb(shape):
        return pl.BlockSpec(shape, lambda i: tuple(0 for _ in shape))

    has_ead_out = eadW is not None
    n_ead = len(ead_list)
    in_specs = ([eb(e.shape[1]) for e in ead_list]
                + [eb(NWAVE), eb(27 * NWAVE),
                   eb(PNORB * NWAVE), eb(PNORB * NWAVE),
                   wb(mpW1.shape), wb(mpW2e.shape),
                   wb(psA.shape), wb(psB.shape), wb(qmat.shape)])
    args = list(ead_list) + [geo, rad, nc0, nc1, mpW1, mpW2e, psA, psB, qmat]
    if has_ead_out:
        in_specs += [wb(eadW[0].shape), wb(eadW[1].shape)]
        args += [eadW[0], eadW[1]]
    outs = [jax.ShapeDtypeStruct((EP, NWAVE), jnp.float32),
            jax.ShapeDtypeStruct((EP, 5 * NWAVE), jnp.float32),
            jax.ShapeDtypeStruct((EP, 4 * NWAVE), jnp.float32),
            jax.ShapeDtypeStruct((EP, 5 * NWAVE), jnp.float32),
            jax.ShapeDtypeStruct((EP, 4 * NWAVE), jnp.float32)]
    out_specs = [eb(NWAVE), eb(5 * NWAVE), eb(4 * NWAVE),
                 eb(5 * NWAVE), eb(4 * NWAVE)]
    if has_ead_out:
        outs.append(jax.ShapeDtypeStruct((EP, 27 * NWAVE), jnp.float32))
        out_specs.append(eb(27 * NWAVE))

    def body(*refs):
        _edge2_body(has_ead_out, n_ead, refs)

    return pl.pallas_call(
        body,
        grid=(EP // BLK,),
        in_specs=in_specs,
        out_specs=tuple(out_specs),
        out_shape=tuple(outs),
    )(*args)


def _expand_blocks(W, blocks):
    H = W.shape[0]
    Wb = W.reshape(H, -1, NWAVE)
    return Wb[:, list(blocks)].reshape(H, len(blocks) * NWAVE)


def kernel(cart, cell, disp_cell, neighlist, celllist, shiftimage,
           center_factor, species, params):
    p = params
    f32 = jnp.float32
    com_spec = jnp.array([[float(i), float(j)] for i in range(NSPEC)
                          for j in range(NSPEC)], dtype=f32)

    symm_cell = (disp_cell + jnp.transpose(disp_cell, (0, 2, 1))) / 2.0
    cell = cell + jnp.einsum('ijk,ikm->ijm', cell, symm_cell)
    symm_cell_n = symm_cell[celllist]
    cart = cart + jnp.einsum('ij,ijk->ik', cart, symm_cell_n)
    cellm = jnp.concatenate(
        [cell.reshape(G, 9), jnp.zeros((G, 7), f32)], axis=1)

    pad_idx = jnp.full((EP - E,), _PAD_NODE, jnp.int32)
    idx0 = jnp.concatenate([neighlist[0], pad_idx]).reshape(_NW, _NCHUNK, CH)
    idx1 = jnp.concatenate([neighlist[1], pad_idx]).reshape(_NW, _NCHUNK, CH)
    spec_idx = species

    node_tab = jnp.concatenate(
        [cart, jnp.zeros((N, 1), f32), spec_idx[:, None].astype(f32),
         celllist[:, None].astype(f32), jnp.zeros((N, 10), f32)], axis=1)
    gs, gd = _sc_gather2(node_tab, idx0, idx1)

    shp = jnp.concatenate(
        [shiftimage, jnp.zeros((3, EP - E), f32)], axis=1)
    shp = jnp.concatenate([shp, jnp.zeros((5, EP), f32)], axis=0)

    pair_spec = _silu(com_spec @ p['ncW1'] + p['ncB1']) @ p['ncW2'] + p['ncB2']
    embt = (_silu(pair_spec @ p['nnW1'] + p['nnB1']) @ p['nnW2']
            + p['nnB2'])
    ieadt = _silu(pair_spec @ p['rwW1']) @ p['rwW2']

    psA_np = np.zeros((NWAVE, 5 * NWAVE), np.float32)
    psB_np = np.zeros((NWAVE, 4 * NWAVE), np.float32)
    for j in range(5):
        psA_np[j, j * NWAVE:(j + 1) * NWAVE] = 1.0
    for j in range(5, PNORB):
        psB_np[j, (j - 5) * NWAVE:(j - 4) * NWAVE] = 1.0
    psA = jnp.asarray(psA_np)
    psB = jnp.asarray(psB_np)
    q_np = np.zeros((PNORB * NWAVE, NWAVE), np.float32)
    for j in range(PNORB):
        for m in range(NWAVE):
            q_np[j * NWAVE + m, m] = 1.0 / np.sqrt(2.0)
    qmat = jnp.asarray(q_np)

    rdW2 = p['rdW2']
    rdW2a = rdW2[:, 3 * NWAVE:5 * NWAVE]
    rdW2bA = _expand_blocks(rdW2, PIDX[:5])
    rdW2bB = _expand_blocks(rdW2, PIDX[5:])
    rxblocks = [r * PRMAXL + PIDX[j] for r in range(3) for j in range(PNORB)]
    e2W2e = _expand_blocks(p['ead2W2'], rxblocks)
    mpW2e = [_expand_blocks(p['mp0W2'], PIDX),
             _expand_blocks(p['mp1W2'], PIDX)]
    ead0W2e = _expand_blocks(p['ead0W2'], rxblocks)

    geo, ead0, wdc, worbA, worbB, rad = _edge_phase1(
        gs, gd, shp, cellm, embt, ieadt, p['rdW1'], rdW2a, rdW2bA, rdW2bB,
        p['ead2W1'], e2W2e, psA, psB)

    wdc_n = _sc_scatter_multi([wdc], idx0)[0]
    density = wdc_n[:, :NWAVE]
    ave = wdc_n[:, NWAVE:NWAVE + 1] + EPS
    corb = jnp.concatenate(
        [_sc_scatter_multi([worbA], idx0)[0],
         _sc_scatter_multi([worbB], idx0)[0]],
        axis=1).reshape(N, PNORB, NWAVE)
    spec_co = (p['spec_coeff'] / np.sqrt(float(NWAVE)))[spec_idx]
    corb = jnp.einsum('ikm,ijk->ijm', spec_co, corb / ave[:, None])

    ead_list = [ead0]
    mpW1s = [p['mp0W1'], p['mp1W1']]
    for it in range(MP_LOOP):
        norm_corb = corb * (1.0 / np.sqrt(float(PRMAXL)))
        ncf = norm_corb.reshape(N, PNORB * NWAVE)
        nc0, nc1 = _sc_gather2(ncf, idx0, idx1)
        eadW = ((p['ead0W1'], ead0W2e) if it < MP_LOOP - 1 else None)
        res = _edge_phase2(ead_list, geo, rad, nc0, nc1,
                           mpW1s[it], mpW2e[it], psA, psB, qmat, eadW)
        ne, orbA, orbB, nworbA, nworbB = res[0], res[1], res[2], res[3], res[4]
        if eadW is not None:
            rad = res[5]
        ead_list = ead_list + [ne]
        sum_orb = jnp.concatenate(
            [_sc_scatter_multi([orbA], idx0)[0],
             _sc_scatter_multi([orbB], idx0)[0]],
            axis=1).reshape(N, PNORB, NWAVE)
        density1 = jnp.sum(sum_orb * norm_corb, axis=1)
        density = jnp.concatenate([density, density1], axis=1)
        sum_new = jnp.concatenate(
            [_sc_scatter_multi([nworbA], idx0)[0],
             _sc_scatter_multi([nworbB], idx0)[0]],
            axis=1).reshape(N, PNORB, NWAVE)
        cc = p['contract_coeff'][it][spec_idx]
        corb = (jnp.einsum('ikm,ijk->ijm', cc[:, 0] / np.sqrt(float(NWAVE)),
                           sum_new / ave[:, None])
                + jnp.einsum('ikm,ijk->ijm', cc[:, 1], corb))
        lmod = jnp.transpose(p['l_coeff'][it][:, spec_idx], (1, 0, 2))
        corb = corb * lmod

    atom_out = (_silu(density @ p['outW1'] + p['outB1']) @ p['outW2']
                + p['outB2'])[:, 0]
    sc = p['scale'].reshape(NSPEC, 2)[spec_idx]
    atom_energy = (atom_out * sc[:, 0] + sc[:, 1]) * center_factor
    energy = jax.ops.segment_sum(atom_energy, celllist, num_segments=G,
                                 indices_are_sorted=True)
    return energy


# speedup vs baseline: 1.5700x; 1.0035x over previous
import functools

import jax
import jax.numpy as jnp
import numpy as np
from jax import lax
from jax.experimental import pallas as pl
from jax.experimental.pallas import tpu as pltpu
from jax.experimental.pallas import tpu_sc as plsc

N = 10000
E = 160000
G = 8
NSPEC = 4
NWAVE = 16
PRMAXL = 3
PNORB = 9
MP_LOOP = 2
CUTOFF = 5.0
PN = 2.0
EPS = 1e-8
PIDX = (0, 1, 1, 1, 2, 2, 2, 2, 2)

EP = 163840
BLK = 2048
CH = 128

_NC = 2
_NS = 16
_NW = _NC * _NS
_PER_TILE = EP // _NW
_NCHUNK = _PER_TILE // CH
_NPAD = N
_NROWS = _NPAD // _NS
_PAD_NODE = N - 1
_ZROWS = 125


def _silu(x):
    return x * jax.nn.sigmoid(x)


def _sc_scatter_multi(vals_list, idx3d):
    nv = len(vals_list)
    Cs = [int(v.shape[1]) for v in vals_list]
    mesh = plsc.VectorSubcoreMesh(core_axis_name="c", subcore_axis_name="s")

    scratch = [pltpu.VMEM((_NCHUNK, CH), jnp.int32)]
    scratch += [pltpu.VMEM((2, CH, C), jnp.float32) for C in Cs]
    scratch += [pltpu.VMEM_SHARED((_NPAD, C), jnp.float32) for C in Cs]
    scratch += [pltpu.SemaphoreType.DMA] * (2 * nv)

    @functools.partial(
        pl.kernel, mesh=mesh,
        compiler_params=pltpu.CompilerParams(use_tc_tiling_on_sc=False),
        out_type=tuple(jax.ShapeDtypeStruct((_NC, _NPAD, C), jnp.float32)
                       for C in Cs),
        scratch_types=scratch,
    )
    def k(*refs):
        i = 0
        vals_hbm = refs[i:i + nv]; i += nv
        idx_hbm = refs[i]; i += 1
        zeros_hbm = refs[i:i + nv]; i += nv
        out_hbm = refs[i:i + nv]; i += nv
        idxv = refs[i]; i += 1
        bufs = refs[i:i + nv]; i += nv
        tabs = refs[i:i + nv]; i += nv
        sems = refs[i:i + 2 * nv]; i += 2 * nv

        c = lax.axis_index("c")
        s = lax.axis_index("s")
        wid = c * _NS + s
        base = wid * _PER_TILE
        for v in range(nv):
            for z in range(_NROWS // _ZROWS):
                pltpu.sync_copy(
                    zeros_hbm[v],
                    tabs[v].at[pl.ds(s * _NROWS + z * _ZROWS, _ZROWS), :])
        plsc.subcore_barrier()
        pltpu.sync_copy(idx_hbm.at[wid], idxv)

        def load(v, j, b):
            return pltpu.async_copy(
                vals_hbm[v].at[pl.ds(base + j * CH, CH), :],
                bufs[v].at[b], sems[2 * v + b])

        for v in range(nv):
            load(v, 0, 0)
            load(v, 1, 1)

        def step(j, b):
            for v in range(nv):
                pltpu.make_async_copy(
                    vals_hbm[v].at[pl.ds(base + j * CH, CH), :],
                    bufs[v].at[b], sems[2 * v + b]).wait()
                pltpu.sync_copy(bufs[v].at[b], tabs[v].at[idxv.at[j]],
                                add=True)

                @pl.when(j + 2 < _NCHUNK)
                def _():
                    load(v, j + 2, b)

        def outer(t, carry):
            step(2 * t, 0)
            step(2 * t + 1, 1)
            return carry

        lax.fori_loop(0, _NCHUNK // 2, outer, 0)
        plsc.subcore_barrier()
        for v in range(nv):
            pltpu.sync_copy(tabs[v].at[pl.ds(s * _NROWS, _NROWS), :],
                            out_hbm[v].at[c].at[pl.ds(s * _NROWS, _NROWS), :])

    zeros = [jnp.zeros((_ZROWS, C), jnp.float32) for C in Cs]
    parts = k(*vals_list, idx3d, *zeros)
    if not isinstance(parts, (tuple, list)):
        parts = (parts,)
    return [part[0, :N] + part[1, :N] for part in parts]


def _sc_gather2(table, idx3d_a, idx3d_b):
    C = table.shape[1]
    mesh = plsc.VectorSubcoreMesh(core_axis_name="c", subcore_axis_name="s")

    @functools.partial(
        pl.kernel, mesh=mesh,
        compiler_params=pltpu.CompilerParams(use_tc_tiling_on_sc=False),
        out_type=(jax.ShapeDtypeStruct((EP, C), jnp.float32),
                  jax.ShapeDtypeStruct((EP, C), jnp.float32)),
        scratch_types=[
            pltpu.VMEM((_NCHUNK, CH), jnp.int32),
            pltpu.VMEM((_NCHUNK, CH), jnp.int32),
            pltpu.VMEM((3, CH, C), jnp.float32),
            pltpu.VMEM((3, CH, C), jnp.float32),
            pltpu.SemaphoreType.DMA,
            pltpu.SemaphoreType.DMA,
            pltpu.SemaphoreType.DMA,
            pltpu.SemaphoreType.DMA,
            pltpu.SemaphoreType.DMA,
            pltpu.SemaphoreType.DMA,
        ],
    )
    def k(tab_hbm, ia_hbm, ib_hbm, outa_hbm, outb_hbm,
          idxa, idxb, bufa, bufb, sa0, sa1, sa2, sb0, sb1, sb2):
        c = lax.axis_index("c")
        s = lax.axis_index("s")
        wid = c * _NS + s
        base = wid * _PER_TILE
        sas = (sa0, sa1, sa2)
        sbs = (sb0, sb1, sb2)
        pltpu.sync_copy(ia_hbm.at[wid], idxa)
        pltpu.sync_copy(ib_hbm.at[wid], idxb)

        def issue(j, b):
            pltpu.async_copy(tab_hbm.at[idxa.at[j]], bufa.at[b], sas[b])
            pltpu.async_copy(tab_hbm.at[idxb.at[j]], bufb.at[b], sbs[b])

        issue(0, 0)
        issue(1, 1)
        issue(2, 2)

        def step(j, b):
            pltpu.make_async_copy(tab_hbm.at[idxa.at[j]], bufa.at[b],
                                  sas[b]).wait()
            pltpu.make_async_copy(tab_hbm.at[idxb.at[j]], bufb.at[b],
                                  sbs[b]).wait()
            pltpu.sync_copy(bufa.at[b],
                            outa_hbm.at[pl.ds(base + j * CH, CH), :])
            pltpu.sync_copy(bufb.at[b],
                            outb_hbm.at[pl.ds(base + j * CH, CH), :])

            @pl.when(j + 3 < _NCHUNK)
            def _():
                issue(j + 3, b)

        def outer(t, carry):
            step(3 * t, 0)
            step(3 * t + 1, 1)
            step(3 * t + 2, 2)
            return carry

        lax.fori_loop(0, _NCHUNK // 3, outer, 0)
        for j in range((_NCHUNK // 3) * 3, _NCHUNK):
            step(j, j % 3)

    return k(table, idx3d_a, idx3d_b)


def _edge1_body(gs_ref, gd_ref, sh_ref, cellm_ref, embt_ref, ieadt_ref,
                rdW1_ref, rdW2a_ref, rdW2bA_ref, rdW2bB_ref,
                e2W1_ref, e2W2e_ref, psA_ref, psB_ref,
                geo_ref, ead_ref, wdc_ref, worbA_ref, worbB_ref, rad_ref):
    gst = jnp.transpose(gs_ref[...])
    gdt = jnp.transpose(gd_ref[...])
    sht = sh_ref[...]
    cellm = cellm_ref[...]
    nedge = gs_ref.shape[0]

    xs, ys, zs = gst[0:1], gst[1:2], gst[2:3]
    xd, yd, zd = gdt[0:1], gdt[1:2], gdt[2:3]
    spec_s, cidx = gst[4:5], gst[5:6]
    spec_d = gdt[4:5]

    oh = [(cidx == float(g)).astype(jnp.float32) for g in range(G)]
    cmv = []
    for q in range(9):
        acc = None
        for g in range(G):
            t = oh[g] * cellm[g:g + 1, q:q + 1]
            acc = t if acc is None else acc + t
        cmv.append(acc)
    sv = [sht[0:1] * cmv[0 + kk] + sht[1:2] * cmv[3 + kk]
          + sht[2:3] * cmv[6 + kk] for kk in range(3)]

    dx = xd - xs + sv[0]
    dy = yd - ys + sv[1]
    dz = zd - zs + sv[2]
    distsq = dx * dx + dy * dy + dz * dz
    nf = (distsq > EPS).astype(jnp.float32)
    dist = jnp.sqrt(distsq + EPS)
    inv = 1.0 / dist
    ux, uy, uz = dx * inv, dy * inv, dz * inv
    s = [jnp.ones_like(ux), ux, uy, uz, ux * uy, uy * uz,
         3.0 * uz * uz - 1.0, uz * ux, ux * ux - uy * uy]
    n1 = ux * ux + uy * uy + uz * uz + EPS
    n2 = (s[4] * s[4] + s[5] * s[5] + s[6] * s[6] + s[7] * s[7]
          + s[8] * s[8] + EPS)
    f = [lax.rsqrt(jnp.ones_like(ux) + EPS),
         jnp.sqrt(3.0) * lax.rsqrt(n1),
         jnp.sqrt(5.0) * lax.rsqrt(n2)]
    sph = [s[j] * f[PIDX[j]] for j in range(PNORB)]

    nd = dist * (1.0 / CUTOFF)
    poly = 1.0 - nd * nd * ((PN + 1.0) * (PN + 2.0) / 2.0
                            - PN * (PN + 2.0) * nd
                            + PN * (PN + 1.0) / 2.0 * nd * nd)
    cut = poly * poly * nf
    pi = spec_s * float(NSPEC) + spec_d

    geo_rows = jnp.concatenate(
        sph + [nd, cut, pi, jnp.zeros((4, nedge), jnp.float32)], axis=0)
    geo = jnp.transpose(geo_rows)
    geo_ref[...] = geo

    nd_c = geo[:, 9:10]
    cut_c = geo[:, 10:11]
    pi_c = geo[:, 11:12]
    lane = lax.broadcasted_iota(
        jnp.int32, (1, NSPEC * NSPEC), 1).astype(jnp.float32)
    ohp = (pi_c == lane).astype(jnp.float32)
    embc = jnp.dot(ohp, embt_ref[...], preferred_element_type=jnp.float32)
    iead = jnp.dot(ohp, ieadt_ref[...], preferred_element_type=jnp.float32)

    smooth = iead * cut_c
    rf = jnp.sinc(nd_c * embc) * cut_c
    radial_func = jnp.concatenate([smooth[:, NWAVE:], rf], axis=1)
    h = _silu(jnp.dot(radial_func, rdW1_ref[...],
                      preferred_element_type=jnp.float32))
    wr_a = jnp.dot(h, rdW2a_ref[...], preferred_element_type=jnp.float32)
    ead = jnp.concatenate([smooth[:, :NWAVE], wr_a[:, NWAVE:]], axis=1)
    ead_ref[...] = ead
    wdc_ref[...] = jnp.concatenate(
        [wr_a[:, :NWAVE], cut_c,
         jnp.zeros((nedge, NWAVE - 1), jnp.float32)], axis=1)

    sphxA = jnp.dot(geo, psA_ref[...], preferred_element_type=jnp.float32)
    sphxB = jnp.dot(geo, psB_ref[...], preferred_element_type=jnp.float32)
    worbA_ref[...] = jnp.dot(h, rdW2bA_ref[...],
                             preferred_element_type=jnp.float32) * sphxA
    worbB_ref[...] = jnp.dot(h, rdW2bB_ref[...],
                             preferred_element_type=jnp.float32) * sphxB

    h2 = _silu(jnp.dot(ead, e2W1_ref[...],
                       preferred_element_type=jnp.float32))
    rad_ref[...] = jnp.dot(h2, e2W2e_ref[...],
                           preferred_element_type=jnp.float32)


def _edge_phase1(gs, gd, shp, cellm, embt, ieadt, rdW1, rdW2a, rdW2bA,
                 rdW2bB, e2W1, e2W2e, psA, psB):
    def eb(c):
        return pl.BlockSpec((BLK, c), lambda i: (i, 0))

    def wb(shape):
        return pl.BlockSpec(shape, lambda i: tuple(0 for _ in shape))

    outs = (
        jax.ShapeDtypeStruct((EP, NWAVE), jnp.float32),
        jax.ShapeDtypeStruct((EP, 2 * NWAVE), jnp.float32),
        jax.ShapeDtypeStruct((EP, 2 * NWAVE), jnp.float32),
        jax.ShapeDtypeStruct((EP, 5 * NWAVE), jnp.float32),
        jax.ShapeDtypeStruct((EP, 4 * NWAVE), jnp.float32),
        jax.ShapeDtypeStruct((EP, 27 * NWAVE), jnp.float32),
    )
    return pl.pallas_call(
        _edge1_body,
        grid=(EP // BLK,),
        in_specs=[eb(16), eb(16),
                  pl.BlockSpec((8, BLK), lambda i: (0, i)),
                  wb(cellm.shape), wb(embt.shape), wb(ieadt.shape),
                  wb(rdW1.shape), wb(rdW2a.shape), wb(rdW2bA.shape),
                  wb(rdW2bB.shape), wb(e2W1.shape), wb(e2W2e.shape),
                  wb(psA.shape), wb(psB.shape)],
        out_specs=(eb(NWAVE), eb(2 * NWAVE), eb(2 * NWAVE),
                   eb(5 * NWAVE), eb(4 * NWAVE), eb(27 * NWAVE)),
        out_shape=outs,
    )(gs, gd, shp, cellm, embt, ieadt, rdW1, rdW2a, rdW2bA, rdW2bB,
      e2W1, e2W2e, psA, psB)


def _edge2_body(has_ead_out, ead_parts, refs):
    i = 0
    eads = []
    for _ in range(ead_parts):
        eads.append(refs[i][...])
        i += 1
    geo = refs[i][...]; i += 1
    rad = refs[i][...]; i += 1
    nc0 = refs[i][...]; i += 1
    nc1 = refs[i][...]; i += 1
    mpW1 = refs[i][...]; i += 1
    mpW2e = refs[i][...]; i += 1
    psA = refs[i][...]; i += 1
    psB = refs[i][...]; i += 1
    qmat = refs[i][...]; i += 1
    if has_ead_out:
        eW1 = refs[i][...]; i += 1
        eW2e = refs[i][...]; i += 1
    ne_ref = refs[i]; i += 1
    orbA_ref = refs[i]; i += 1
    orbB_ref = refs[i]; i += 1
    nworbA_ref = refs[i]; i += 1
    nworbB_ref = refs[i]; i += 1
    if has_ead_out:
        radnew_ref = refs[i]; i += 1

    W = PNORB * NWAVE
    sphxA = jnp.dot(geo, psA, preferred_element_type=jnp.float32)
    sphxB = jnp.dot(geo, psB, preferred_element_type=jnp.float32)
    r0A, r0B = rad[:, 0:80], rad[:, 80:W]
    r1A, r1B = rad[:, W:W + 80], rad[:, W + 80:2 * W]
    r2A, r2B = rad[:, 2 * W:2 * W + 80], rad[:, 2 * W + 80:3 * W]
    aoA = (r0A * nc0[:, :80] + r1A * nc1[:, :80]) * sphxA
    aoB = (r0B * nc0[:, 80:] + r1B * nc1[:, 80:]) * sphxB
    ne = (jnp.dot(aoA, qmat[:80], preferred_element_type=jnp.float32)
          + jnp.dot(aoB, qmat[80:], preferred_element_type=jnp.float32))
    ne_ref[...] = ne
    orbA_ref[...] = r2A * sphxA
    orbB_ref[...] = r2B * sphxB

    ead_cat = jnp.concatenate(eads + [ne], axis=1)
    h = _silu(jnp.dot(ead_cat, mpW1, preferred_element_type=jnp.float32))
    nwA = jnp.dot(h, mpW2e[:, :80], preferred_element_type=jnp.float32)
    nwB = jnp.dot(h, mpW2e[:, 80:], preferred_element_type=jnp.float32)
    nworbA_ref[...] = nwA * sphxA
    nworbB_ref[...] = nwB * sphxB
    if has_ead_out:
        h2 = _silu(jnp.dot(ead_cat, eW1, preferred_element_type=jnp.float32))
        radnew_ref[...] = jnp.dot(h2, eW2e,
                                  preferred_element_type=jnp.float32)


def _edge_phase2(ead_list, geo, rad, nc0, nc1, mpW1, mpW2e, psA, psB, qmat,
                 eadW=None):
    def eb(c):
        return pl.BlockSpec((BLK, c), lambda i: (i, 0))

    def wb(shape):
        return pl.BlockSpec(shape, lambda i: tuple(0 for _ in shape))

    has_ead_out = eadW is not None
    n_ead = len(ead_list)
    in_specs = ([eb(e.shape[1]) for e in ead_list]
                + [eb(NWAVE), eb(27 * NWAVE),
                   eb(PNORB * NWAVE), eb(PNORB * NWAVE),
                   wb(mpW1.shape), wb(mpW2e.shape),
                   wb(psA.shape), wb(psB.shape), wb(qmat.shape)])
    args = list(ead_list) + [geo, rad, nc0, nc1, mpW1, mpW2e, psA, psB, qmat]
    if has_ead_out:
        in_specs += [wb(eadW[0].shape), wb(eadW[1].shape)]
        args += [eadW[0], eadW[1]]
    outs = [jax.ShapeDtypeStruct((EP, NWAVE), jnp.float32),
            jax.ShapeDtypeStruct((EP, 5 * NWAVE), jnp.float32),
            jax.ShapeDtypeStruct((EP, 4 * NWAVE), jnp.float32),
            jax.ShapeDtypeStruct((EP, 5 * NWAVE), jnp.float32),
            jax.ShapeDtypeStruct((EP, 4 * NWAVE), jnp.float32)]
    out_specs = [eb(NWAVE), eb(5 * NWAVE), eb(4 * NWAVE),
                 eb(5 * NWAVE), eb(4 * NWAVE)]
    if has_ead_out:
        outs.append(jax.ShapeDtypeStruct((EP, 27 * NWAVE), jnp.float32))
        out_specs.append(eb(27 * NWAVE))

    def body(*refs):
        _edge2_body(has_ead_out, n_ead, refs)

    return pl.pallas_call(
        body,
        grid=(EP // BLK,),
        in_specs=in_specs,
        out_specs=tuple(out_specs),
        out_shape=tuple(outs),
    )(*args)


def _expand_blocks(W, blocks):
    H = W.shape[0]
    Wb = W.reshape(H, -1, NWAVE)
    return Wb[:, list(blocks)].reshape(H, len(blocks) * NWAVE)


def kernel(cart, cell, disp_cell, neighlist, celllist, shiftimage,
           center_factor, species, params):
    p = params
    f32 = jnp.float32
    com_spec = jnp.array([[float(i), float(j)] for i in range(NSPEC)
                          for j in range(NSPEC)], dtype=f32)

    symm_cell = (disp_cell + jnp.transpose(disp_cell, (0, 2, 1))) / 2.0
    cell = cell + jnp.einsum('ijk,ikm->ijm', cell, symm_cell)
    symm_cell_n = symm_cell[celllist]
    cart = cart + jnp.einsum('ij,ijk->ik', cart, symm_cell_n)
    cellm = jnp.concatenate(
        [cell.reshape(G, 9), jnp.zeros((G, 7), f32)], axis=1)

    pad_idx = jnp.full((EP - E,), _PAD_NODE, jnp.int32)
    idx0 = jnp.concatenate([neighlist[0], pad_idx]).reshape(_NW, _NCHUNK, CH)
    idx1 = jnp.concatenate([neighlist[1], pad_idx]).reshape(_NW, _NCHUNK, CH)
    spec_idx = species

    node_tab = jnp.concatenate(
        [cart, jnp.zeros((N, 1), f32), spec_idx[:, None].astype(f32),
         celllist[:, None].astype(f32), jnp.zeros((N, 10), f32)], axis=1)
    gs, gd = _sc_gather2(node_tab, idx0, idx1)

    shp = jnp.concatenate(
        [shiftimage, jnp.zeros((3, EP - E), f32)], axis=1)
    shp = jnp.concatenate([shp, jnp.zeros((5, EP), f32)], axis=0)

    pair_spec = _silu(com_spec @ p['ncW1'] + p['ncB1']) @ p['ncW2'] + p['ncB2']
    embt = (_silu(pair_spec @ p['nnW1'] + p['nnB1']) @ p['nnW2']
            + p['nnB2'])
    ieadt = _silu(pair_spec @ p['rwW1']) @ p['rwW2']

    psA_np = np.zeros((NWAVE, 5 * NWAVE), np.float32)
    psB_np = np.zeros((NWAVE, 4 * NWAVE), np.float32)
    for j in range(5):
        psA_np[j, j * NWAVE:(j + 1) * NWAVE] = 1.0
    for j in range(5, PNORB):
        psB_np[j, (j - 5) * NWAVE:(j - 4) * NWAVE] = 1.0
    psA = jnp.asarray(psA_np)
    psB = jnp.asarray(psB_np)
    q_np = np.zeros((PNORB * NWAVE, NWAVE), np.float32)
    for j in range(PNORB):
        for m in range(NWAVE):
            q_np[j * NWAVE + m, m] = 1.0 / np.sqrt(2.0)
    qmat = jnp.asarray(q_np)

    rdW2 = p['rdW2']
    rdW2a = rdW2[:, 3 * NWAVE:5 * NWAVE]
    rdW2bA = _expand_blocks(rdW2, PIDX[:5])
    rdW2bB = _expand_blocks(rdW2, PIDX[5:])
    rxblocks = [r * PRMAXL + PIDX[j] for r in range(3) for j in range(PNORB)]
    e2W2e = _expand_blocks(p['ead2W2'], rxblocks)
    mpW2e = [_expand_blocks(p['mp0W2'], PIDX),
             _expand_blocks(p['mp1W2'], PIDX)]
    ead0W2e = _expand_blocks(p['ead0W2'], rxblocks)

    geo, ead0, wdc, worbA, worbB, rad = _edge_phase1(
        gs, gd, shp, cellm, embt, ieadt, p['rdW1'], rdW2a, rdW2bA, rdW2bB,
        p['ead2W1'], e2W2e, psA, psB)

    wdc_n = _sc_scatter_multi([wdc], idx0)[0]
    density = wdc_n[:, :NWAVE]
    ave = wdc_n[:, NWAVE:NWAVE + 1] + EPS
    corb = jnp.concatenate(
        [_sc_scatter_multi([worbA], idx0)[0],
         _sc_scatter_multi([worbB], idx0)[0]],
        axis=1).reshape(N, PNORB, NWAVE)
    spec_co = (p['spec_coeff'] / np.sqrt(float(NWAVE)))[spec_idx]
    corb = jnp.einsum('ikm,ijk->ijm', spec_co, corb / ave[:, None])

    ead_list = [ead0]
    mpW1s = [p['mp0W1'], p['mp1W1']]
    for it in range(MP_LOOP):
        norm_corb = corb * (1.0 / np.sqrt(float(PRMAXL)))
        ncf = norm_corb.reshape(N, PNORB * NWAVE)
        nc0, nc1 = _sc_gather2(ncf, idx0, idx1)
        eadW = ((p['ead0W1'], ead0W2e) if it < MP_LOOP - 1 else None)
        res = _edge_phase2(ead_list, geo, rad, nc0, nc1,
                           mpW1s[it], mpW2e[it], psA, psB, qmat, eadW)
        ne, orbA, orbB, nworbA, nworbB = res[0], res[1], res[2], res[3], res[4]
        if eadW is not None:
            rad = res[5]
        ead_list = ead_list + [ne]
        sum_orb = jnp.concatenate(
            [_sc_scatter_multi([orbA], idx0)[0],
             _sc_scatter_multi([orbB], idx0)[0]],
            axis=1).reshape(N, PNORB, NWAVE)
        density1 = jnp.sum(sum_orb * norm_corb, axis=1)
        density = jnp.concatenate([density, density1], axis=1)
        sum_new = jnp.concatenate(
            [_sc_scatter_multi([nworbA], idx0)[0],
             _sc_scatter_multi([nworbB], idx0)[0]],
            axis=1).reshape(N, PNORB, NWAVE)
        cc = p['contract_coeff'][it][spec_idx]
        corb = (jnp.einsum('ikm,ijk->ijm', cc[:, 0] / np.sqrt(float(NWAVE)),
                           sum_new / ave[:, None])
                + jnp.einsum('ikm,ijk->ijm', cc[:, 1], corb))
        lmod = jnp.transpose(p['l_coeff'][it][:, spec_idx], (1, 0, 2))
        corb = corb * lmod

    atom_out = (_silu(density @ p['outW1'] + p['outB1']) @ p['outW2']
                + p['outB2'])[:, 0]
    sc = p['scale'].reshape(NSPEC, 2)[spec_idx]
    atom_energy = (atom_out * sc[:, 0] + sc[:, 1]) * center_factor
    energy = jax.ops.segment_sum(atom_energy, celllist, num_segments=G,
                                 indices_are_sorted=True)
    return energy


# speedup vs baseline: 1.5891x; 1.0121x over previous
import functools

import jax
import jax.numpy as jnp
import numpy as np
from jax import lax
from jax.experimental import pallas as pl
from jax.experimental.pallas import tpu as pltpu
from jax.experimental.pallas import tpu_sc as plsc

N = 10000
E = 160000
G = 8
NSPEC = 4
NWAVE = 16
PRMAXL = 3
PNORB = 9
MP_LOOP = 2
CUTOFF = 5.0
PN = 2.0
EPS = 1e-8
PIDX = (0, 1, 1, 1, 2, 2, 2, 2, 2)

EP = 163840
BLK = 2048
CH = 128

_NC = 2
_NS = 16
_NW = _NC * _NS
_PER_TILE = EP // _NW
_NCHUNK = _PER_TILE // CH
_NPAD = N
_NROWS = _NPAD // _NS
_PAD_NODE = N - 1
_ZROWS = 125


def _silu(x):
    return x * jax.nn.sigmoid(x)


def _sc_scatter_multi(vals_list, idx3d):
    nv = len(vals_list)
    Cs = [int(v.shape[1]) for v in vals_list]
    mesh = plsc.VectorSubcoreMesh(core_axis_name="c", subcore_axis_name="s")

    scratch = [pltpu.VMEM((_NCHUNK, CH), jnp.int32)]
    scratch += [pltpu.VMEM((2, CH, C), jnp.float32) for C in Cs]
    scratch += [pltpu.VMEM_SHARED((_NPAD, C), jnp.float32) for C in Cs]
    scratch += [pltpu.SemaphoreType.DMA] * (2 * nv)

    @functools.partial(
        pl.kernel, mesh=mesh,
        compiler_params=pltpu.CompilerParams(use_tc_tiling_on_sc=False),
        out_type=tuple(jax.ShapeDtypeStruct((_NC, _NPAD, C), jnp.float32)
                       for C in Cs),
        scratch_types=scratch,
    )
    def k(*refs):
        i = 0
        vals_hbm = refs[i:i + nv]; i += nv
        idx_hbm = refs[i]; i += 1
        zeros_hbm = refs[i:i + nv]; i += nv
        out_hbm = refs[i:i + nv]; i += nv
        idxv = refs[i]; i += 1
        bufs = refs[i:i + nv]; i += nv
        tabs = refs[i:i + nv]; i += nv
        sems = refs[i:i + 2 * nv]; i += 2 * nv

        c = lax.axis_index("c")
        s = lax.axis_index("s")
        wid = c * _NS + s
        base = wid * _PER_TILE
        for v in range(nv):
            for z in range(_NROWS // _ZROWS):
                pltpu.sync_copy(
                    zeros_hbm[v],
                    tabs[v].at[pl.ds(s * _NROWS + z * _ZROWS, _ZROWS), :])
        plsc.subcore_barrier()
        pltpu.sync_copy(idx_hbm.at[wid], idxv)

        def load(v, j, b):
            return pltpu.async_copy(
                vals_hbm[v].at[pl.ds(base + j * CH, CH), :],
                bufs[v].at[b], sems[2 * v + b])

        for v in range(nv):
            load(v, 0, 0)
            load(v, 1, 1)

        def step(j, b):
            for v in range(nv):
                pltpu.make_async_copy(
                    vals_hbm[v].at[pl.ds(base + j * CH, CH), :],
                    bufs[v].at[b], sems[2 * v + b]).wait()
                pltpu.sync_copy(bufs[v].at[b], tabs[v].at[idxv.at[j]],
                                add=True)

                @pl.when(j + 2 < _NCHUNK)
                def _():
                    load(v, j + 2, b)

        def outer(t, carry):
            step(2 * t, 0)
            step(2 * t + 1, 1)
            return carry

        lax.fori_loop(0, _NCHUNK // 2, outer, 0)
        plsc.subcore_barrier()
        for v in range(nv):
            pltpu.sync_copy(tabs[v].at[pl.ds(s * _NROWS, _NROWS), :],
                            out_hbm[v].at[c].at[pl.ds(s * _NROWS, _NROWS), :])

    zeros = [jnp.zeros((_ZROWS, C), jnp.float32) for C in Cs]
    parts = k(*vals_list, idx3d, *zeros)
    if not isinstance(parts, (tuple, list)):
        parts = (parts,)
    return [part[0, :N] + part[1, :N] for part in parts]


def _sc_gather2(table, idx3d_a, idx3d_b):
    C = table.shape[1]
    mesh = plsc.VectorSubcoreMesh(core_axis_name="c", subcore_axis_name="s")

    @functools.partial(
        pl.kernel, mesh=mesh,
        compiler_params=pltpu.CompilerParams(use_tc_tiling_on_sc=False),
        out_type=(jax.ShapeDtypeStruct((EP, C), jnp.float32),
                  jax.ShapeDtypeStruct((EP, C), jnp.float32)),
        scratch_types=[
            pltpu.VMEM((_NCHUNK, CH), jnp.int32),
            pltpu.VMEM((_NCHUNK, CH), jnp.int32),
            pltpu.VMEM((3, CH, C), jnp.float32),
            pltpu.VMEM((3, CH, C), jnp.float32),
            pltpu.SemaphoreType.DMA,
            pltpu.SemaphoreType.DMA,
            pltpu.SemaphoreType.DMA,
            pltpu.SemaphoreType.DMA,
            pltpu.SemaphoreType.DMA,
            pltpu.SemaphoreType.DMA,
        ],
    )
    def k(tab_hbm, ia_hbm, ib_hbm, outa_hbm, outb_hbm,
          idxa, idxb, bufa, bufb, sa0, sa1, sa2, sb0, sb1, sb2):
        c = lax.axis_index("c")
        s = lax.axis_index("s")
        wid = c * _NS + s
        base = wid * _PER_TILE
        sas = (sa0, sa1, sa2)
        sbs = (sb0, sb1, sb2)
        pltpu.sync_copy(ia_hbm.at[wid], idxa)
        pltpu.sync_copy(ib_hbm.at[wid], idxb)

        def issue(j, b):
            pltpu.async_copy(tab_hbm.at[idxa.at[j]], bufa.at[b], sas[b])
            pltpu.async_copy(tab_hbm.at[idxb.at[j]], bufb.at[b], sbs[b])

        issue(0, 0)
        issue(1, 1)
        issue(2, 2)

        def step(j, b):
            pltpu.make_async_copy(tab_hbm.at[idxa.at[j]], bufa.at[b],
                                  sas[b]).wait()
            pltpu.make_async_copy(tab_hbm.at[idxb.at[j]], bufb.at[b],
                                  sbs[b]).wait()
            pltpu.sync_copy(bufa.at[b],
                            outa_hbm.at[pl.ds(base + j * CH, CH), :])
            pltpu.sync_copy(bufb.at[b],
                            outb_hbm.at[pl.ds(base + j * CH, CH), :])

            @pl.when(j + 3 < _NCHUNK)
            def _():
                issue(j + 3, b)

        def outer(t, carry):
            step(3 * t, 0)
            step(3 * t + 1, 1)
            step(3 * t + 2, 2)
            return carry

        lax.fori_loop(0, _NCHUNK // 3, outer, 0)
        for j in range((_NCHUNK // 3) * 3, _NCHUNK):
            step(j, j % 3)

    return k(table, idx3d_a, idx3d_b)


def _edge1_body(gs_ref, gd_ref, sh_ref, cellm_ref, embt_ref, ieadt_ref,
                rdW1_ref, rdW2a_ref, rdW2bA_ref, rdW2bB_ref,
                e2W1_ref, e2W2e_ref, psA_ref, psB_ref,
                geo_ref, ead_ref, wdc_ref, worbA_ref, worbB_ref, rad_ref):
    gst = jnp.transpose(gs_ref[...])
    gdt = jnp.transpose(gd_ref[...])
    sht = sh_ref[...]
    cellm = cellm_ref[...]
    nedge = gs_ref.shape[0]

    xs, ys, zs = gst[0:1], gst[1:2], gst[2:3]
    xd, yd, zd = gdt[0:1], gdt[1:2], gdt[2:3]
    spec_s, cidx = gst[4:5], gst[5:6]
    spec_d = gdt[4:5]

    oh = [(cidx == float(g)).astype(jnp.float32) for g in range(G)]
    cmv = []
    for q in range(9):
        acc = None
        for g in range(G):
            t = oh[g] * cellm[g:g + 1, q:q + 1]
            acc = t if acc is None else acc + t
        cmv.append(acc)
    sv = [sht[0:1] * cmv[0 + kk] + sht[1:2] * cmv[3 + kk]
          + sht[2:3] * cmv[6 + kk] for kk in range(3)]

    dx = xd - xs + sv[0]
    dy = yd - ys + sv[1]
    dz = zd - zs + sv[2]
    distsq = dx * dx + dy * dy + dz * dz
    nf = (distsq > EPS).astype(jnp.float32)
    dist = jnp.sqrt(distsq + EPS)
    inv = 1.0 / dist
    ux, uy, uz = dx * inv, dy * inv, dz * inv
    s = [jnp.ones_like(ux), ux, uy, uz, ux * uy, uy * uz,
         3.0 * uz * uz - 1.0, uz * ux, ux * ux - uy * uy]
    n1 = ux * ux + uy * uy + uz * uz + EPS
    n2 = (s[4] * s[4] + s[5] * s[5] + s[6] * s[6] + s[7] * s[7]
          + s[8] * s[8] + EPS)
    f = [lax.rsqrt(jnp.ones_like(ux) + EPS),
         jnp.sqrt(3.0) * lax.rsqrt(n1),
         jnp.sqrt(5.0) * lax.rsqrt(n2)]
    sph = [s[j] * f[PIDX[j]] for j in range(PNORB)]

    nd = dist * (1.0 / CUTOFF)
    poly = 1.0 - nd * nd * ((PN + 1.0) * (PN + 2.0) / 2.0
                            - PN * (PN + 2.0) * nd
                            + PN * (PN + 1.0) / 2.0 * nd * nd)
    cut = poly * poly * nf
    pi = spec_s * float(NSPEC) + spec_d

    geo_rows = jnp.concatenate(
        sph + [nd, cut, pi, jnp.zeros((4, nedge), jnp.float32)], axis=0)
    geo = jnp.transpose(geo_rows)
    geo_ref[...] = geo

    nd_c = geo[:, 9:10]
    cut_c = geo[:, 10:11]
    pi_c = geo[:, 11:12]
    lane = lax.broadcasted_iota(
        jnp.int32, (1, NSPEC * NSPEC), 1).astype(jnp.float32)
    ohp = (pi_c == lane).astype(jnp.float32)
    embc = jnp.dot(ohp, embt_ref[...], preferred_element_type=jnp.float32)
    iead = jnp.dot(ohp, ieadt_ref[...], preferred_element_type=jnp.float32)

    smooth = iead * cut_c
    rf = jnp.sinc(nd_c * embc) * cut_c
    radial_func = jnp.concatenate([smooth[:, NWAVE:], rf], axis=1)
    h = _silu(jnp.dot(radial_func, rdW1_ref[...],
                      preferred_element_type=jnp.float32))
    wr_a = jnp.dot(h, rdW2a_ref[...], preferred_element_type=jnp.float32)
    ead = jnp.concatenate([smooth[:, :NWAVE], wr_a[:, NWAVE:]], axis=1)
    ead_ref[...] = ead
    wdc_ref[...] = jnp.concatenate(
        [wr_a[:, :NWAVE], cut_c,
         jnp.zeros((nedge, NWAVE - 1), jnp.float32)], axis=1)

    sphxA = jnp.dot(geo, psA_ref[...], preferred_element_type=jnp.float32)
    sphxB = jnp.dot(geo, psB_ref[...], preferred_element_type=jnp.float32)
    worbA_ref[...] = jnp.dot(h, rdW2bA_ref[...],
                             preferred_element_type=jnp.float32) * sphxA
    worbB_ref[...] = jnp.dot(h, rdW2bB_ref[...],
                             preferred_element_type=jnp.float32) * sphxB

    h2 = _silu(jnp.dot(ead, e2W1_ref[...],
                       preferred_element_type=jnp.float32))
    rad_ref[...] = jnp.dot(h2, e2W2e_ref[...],
                           preferred_element_type=jnp.float32)


def _edge_phase1(gs, gd, shp, cellm, embt, ieadt, rdW1, rdW2a, rdW2bA,
                 rdW2bB, e2W1, e2W2e, psA, psB):
    def eb(c):
        return pl.BlockSpec((BLK, c), lambda i: (i, 0))

    def wb(shape):
        return pl.BlockSpec(shape, lambda i: tuple(0 for _ in shape))

    outs = (
        jax.ShapeDtypeStruct((EP, NWAVE), jnp.float32),
        jax.ShapeDtypeStruct((EP, 2 * NWAVE), jnp.float32),
        jax.ShapeDtypeStruct((EP, 2 * NWAVE), jnp.float32),
        jax.ShapeDtypeStruct((EP, 5 * NWAVE), jnp.float32),
        jax.ShapeDtypeStruct((EP, 4 * NWAVE), jnp.float32),
        jax.ShapeDtypeStruct((EP, 27 * NWAVE), jnp.float32),
    )
    return pl.pallas_call(
        _edge1_body,
        grid=(EP // BLK,),
        in_specs=[eb(16), eb(16),
                  pl.BlockSpec((8, BLK), lambda i: (0, i)),
                  wb(cellm.shape), wb(embt.shape), wb(ieadt.shape),
                  wb(rdW1.shape), wb(rdW2a.shape), wb(rdW2bA.shape),
                  wb(rdW2bB.shape), wb(e2W1.shape), wb(e2W2e.shape),
                  wb(psA.shape), wb(psB.shape)],
        out_specs=(eb(NWAVE), eb(2 * NWAVE), eb(2 * NWAVE),
                   eb(5 * NWAVE), eb(4 * NWAVE), eb(27 * NWAVE)),
        out_shape=outs,
    )(gs, gd, shp, cellm, embt, ieadt, rdW1, rdW2a, rdW2bA, rdW2bB,
      e2W1, e2W2e, psA, psB)


def _edge2_body(has_ead_out, has_nworb, ead_parts, refs):
    i = 0
    eads = []
    for _ in range(ead_parts):
        eads.append(refs[i][...])
        i += 1
    geo = refs[i][...]; i += 1
    rad = refs[i][...]; i += 1
    nc0 = refs[i][...]; i += 1
    nc1 = refs[i][...]; i += 1
    if has_nworb:
        mpW1 = refs[i][...]; i += 1
        mpW2e = refs[i][...]; i += 1
    psA = refs[i][...]; i += 1
    psB = refs[i][...]; i += 1
    qmat = refs[i][...]; i += 1
    if has_ead_out:
        eW1 = refs[i][...]; i += 1
        eW2e = refs[i][...]; i += 1
    ne_ref = refs[i]; i += 1
    orbA_ref = refs[i]; i += 1
    orbB_ref = refs[i]; i += 1
    if has_nworb:
        nworbA_ref = refs[i]; i += 1
        nworbB_ref = refs[i]; i += 1
    if has_ead_out:
        radnew_ref = refs[i]; i += 1

    W = PNORB * NWAVE
    sphxA = jnp.dot(geo, psA, preferred_element_type=jnp.float32)
    sphxB = jnp.dot(geo, psB, preferred_element_type=jnp.float32)
    r0A, r0B = rad[:, 0:80], rad[:, 80:W]
    r1A, r1B = rad[:, W:W + 80], rad[:, W + 80:2 * W]
    r2A, r2B = rad[:, 2 * W:2 * W + 80], rad[:, 2 * W + 80:3 * W]
    aoA = (r0A * nc0[:, :80] + r1A * nc1[:, :80]) * sphxA
    aoB = (r0B * nc0[:, 80:] + r1B * nc1[:, 80:]) * sphxB
    ne = (jnp.dot(aoA, qmat[:80], preferred_element_type=jnp.float32)
          + jnp.dot(aoB, qmat[80:], preferred_element_type=jnp.float32))
    ne_ref[...] = ne
    orbA_ref[...] = r2A * sphxA
    orbB_ref[...] = r2B * sphxB

    if has_nworb:
        ead_cat = jnp.concatenate(eads + [ne], axis=1)
        h = _silu(jnp.dot(ead_cat, mpW1,
                          preferred_element_type=jnp.float32))
        nwA = jnp.dot(h, mpW2e[:, :80], preferred_element_type=jnp.float32)
        nwB = jnp.dot(h, mpW2e[:, 80:], preferred_element_type=jnp.float32)
        nworbA_ref[...] = nwA * sphxA
        nworbB_ref[...] = nwB * sphxB
    if has_ead_out:
        ead_cat = jnp.concatenate(eads + [ne], axis=1)
        h2 = _silu(jnp.dot(ead_cat, eW1, preferred_element_type=jnp.float32))
        radnew_ref[...] = jnp.dot(h2, eW2e,
                                  preferred_element_type=jnp.float32)


def _edge_phase2(ead_list, geo, rad, nc0, nc1, mpW=None, psA=None, psB=None,
                 qmat=None, eadW=None):
    def eb(c):
        return pl.BlockSpec((BLK, c), lambda i: (i, 0))

    def wb(shape):
        return pl.BlockSpec(shape, lambda i: tuple(0 for _ in shape))

    has_ead_out = eadW is not None
    has_nworb = mpW is not None
    n_ead = len(ead_list)
    in_specs = ([eb(e.shape[1]) for e in ead_list]
                + [eb(NWAVE), eb(27 * NWAVE),
                   eb(PNORB * NWAVE), eb(PNORB * NWAVE)])
    args = list(ead_list) + [geo, rad, nc0, nc1]
    if has_nworb:
        in_specs += [wb(mpW[0].shape), wb(mpW[1].shape)]
        args += [mpW[0], mpW[1]]
    in_specs += [wb(psA.shape), wb(psB.shape), wb(qmat.shape)]
    args += [psA, psB, qmat]
    if has_ead_out:
        in_specs += [wb(eadW[0].shape), wb(eadW[1].shape)]
        args += [eadW[0], eadW[1]]
    outs = [jax.ShapeDtypeStruct((EP, NWAVE), jnp.float32),
            jax.ShapeDtypeStruct((EP, 5 * NWAVE), jnp.float32),
            jax.ShapeDtypeStruct((EP, 4 * NWAVE), jnp.float32)]
    out_specs = [eb(NWAVE), eb(5 * NWAVE), eb(4 * NWAVE)]
    if has_nworb:
        outs += [jax.ShapeDtypeStruct((EP, 5 * NWAVE), jnp.float32),
                 jax.ShapeDtypeStruct((EP, 4 * NWAVE), jnp.float32)]
        out_specs += [eb(5 * NWAVE), eb(4 * NWAVE)]
    if has_ead_out:
        outs.append(jax.ShapeDtypeStruct((EP, 27 * NWAVE), jnp.float32))
        out_specs.append(eb(27 * NWAVE))

    def body(*refs):
        _edge2_body(has_ead_out, has_nworb, n_ead, refs)

    return pl.pallas_call(
        body,
        grid=(EP // BLK,),
        in_specs=in_specs,
        out_specs=tuple(out_specs),
        out_shape=tuple(outs),
    )(*args)


def _expand_blocks(W, blocks):
    H = W.shape[0]
    Wb = W.reshape(H, -1, NWAVE)
    return Wb[:, list(blocks)].reshape(H, len(blocks) * NWAVE)


def kernel(cart, cell, disp_cell, neighlist, celllist, shiftimage,
           center_factor, species, params):
    p = params
    f32 = jnp.float32
    com_spec = jnp.array([[float(i), float(j)] for i in range(NSPEC)
                          for j in range(NSPEC)], dtype=f32)

    symm_cell = (disp_cell + jnp.transpose(disp_cell, (0, 2, 1))) / 2.0
    cell = cell + jnp.einsum('ijk,ikm->ijm', cell, symm_cell)
    symm_cell_n = symm_cell[celllist]
    cart = cart + jnp.einsum('ij,ijk->ik', cart, symm_cell_n)
    cellm = jnp.concatenate(
        [cell.reshape(G, 9), jnp.zeros((G, 7), f32)], axis=1)

    pad_idx = jnp.full((EP - E,), _PAD_NODE, jnp.int32)
    idx0 = jnp.concatenate([neighlist[0], pad_idx]).reshape(_NW, _NCHUNK, CH)
    idx1 = jnp.concatenate([neighlist[1], pad_idx]).reshape(_NW, _NCHUNK, CH)
    spec_idx = species

    node_tab = jnp.concatenate(
        [cart, jnp.zeros((N, 1), f32), spec_idx[:, None].astype(f32),
         celllist[:, None].astype(f32), jnp.zeros((N, 10), f32)], axis=1)
    gs, gd = _sc_gather2(node_tab, idx0, idx1)

    shp = jnp.concatenate(
        [shiftimage, jnp.zeros((3, EP - E), f32)], axis=1)
    shp = jnp.concatenate([shp, jnp.zeros((5, EP), f32)], axis=0)

    pair_spec = _silu(com_spec @ p['ncW1'] + p['ncB1']) @ p['ncW2'] + p['ncB2']
    embt = (_silu(pair_spec @ p['nnW1'] + p['nnB1']) @ p['nnW2']
            + p['nnB2'])
    ieadt = _silu(pair_spec @ p['rwW1']) @ p['rwW2']

    psA_np = np.zeros((NWAVE, 5 * NWAVE), np.float32)
    psB_np = np.zeros((NWAVE, 4 * NWAVE), np.float32)
    for j in range(5):
        psA_np[j, j * NWAVE:(j + 1) * NWAVE] = 1.0
    for j in range(5, PNORB):
        psB_np[j, (j - 5) * NWAVE:(j - 4) * NWAVE] = 1.0
    psA = jnp.asarray(psA_np)
    psB = jnp.asarray(psB_np)
    q_np = np.zeros((PNORB * NWAVE, NWAVE), np.float32)
    for j in range(PNORB):
        for m in range(NWAVE):
            q_np[j * NWAVE + m, m] = 1.0 / np.sqrt(2.0)
    qmat = jnp.asarray(q_np)

    rdW2 = p['rdW2']
    rdW2a = rdW2[:, 3 * NWAVE:5 * NWAVE]
    rdW2bA = _expand_blocks(rdW2, PIDX[:5])
    rdW2bB = _expand_blocks(rdW2, PIDX[5:])
    rxblocks = [r * PRMAXL + PIDX[j] for r in range(3) for j in range(PNORB)]
    e2W2e = _expand_blocks(p['ead2W2'], rxblocks)
    mpW2e = [_expand_blocks(p['mp0W2'], PIDX),
             _expand_blocks(p['mp1W2'], PIDX)]
    ead0W2e = _expand_blocks(p['ead0W2'], rxblocks)

    geo, ead0, wdc, worbA, worbB, rad = _edge_phase1(
        gs, gd, shp, cellm, embt, ieadt, p['rdW1'], rdW2a, rdW2bA, rdW2bB,
        p['ead2W1'], e2W2e, psA, psB)

    wdc_n = _sc_scatter_multi([wdc], idx0)[0]
    density = wdc_n[:, :NWAVE]
    ave = wdc_n[:, NWAVE:NWAVE + 1] + EPS
    corb = jnp.concatenate(
        [_sc_scatter_multi([worbA], idx0)[0],
         _sc_scatter_multi([worbB], idx0)[0]],
        axis=1).reshape(N, PNORB, NWAVE)
    spec_co = (p['spec_coeff'] / np.sqrt(float(NWAVE)))[spec_idx]
    corb = jnp.einsum('ikm,ijk->ijm', spec_co, corb / ave[:, None])

    ead_list = [ead0]
    mpW1s = [p['mp0W1'], p['mp1W1']]
    for it in range(MP_LOOP):
        last = it == MP_LOOP - 1
        norm_corb = corb * (1.0 / np.sqrt(float(PRMAXL)))
        ncf = norm_corb.reshape(N, PNORB * NWAVE)
        nc0, nc1 = _sc_gather2(ncf, idx0, idx1)
        eadW = None if last else (p['ead0W1'], ead0W2e)
        mpW = None if last else (mpW1s[it], mpW2e[it])
        res = _edge_phase2(ead_list, geo, rad, nc0, nc1,
                           mpW, psA, psB, qmat, eadW)
        ne, orbA, orbB = res[0], res[1], res[2]
        ead_list = ead_list + [ne]
        sum_orb = jnp.concatenate(
            [_sc_scatter_multi([orbA], idx0)[0],
             _sc_scatter_multi([orbB], idx0)[0]],
            axis=1).reshape(N, PNORB, NWAVE)
        density1 = jnp.sum(sum_orb * norm_corb, axis=1)
        density = jnp.concatenate([density, density1], axis=1)
        if last:
            break
        nworbA, nworbB, rad = res[3], res[4], res[5]
        sum_new = jnp.concatenate(
            [_sc_scatter_multi([nworbA], idx0)[0],
             _sc_scatter_multi([nworbB], idx0)[0]],
            axis=1).reshape(N, PNORB, NWAVE)
        cc = p['contract_coeff'][it][spec_idx]
        corb = (jnp.einsum('ikm,ijk->ijm', cc[:, 0] / np.sqrt(float(NWAVE)),
                           sum_new / ave[:, None])
                + jnp.einsum('ikm,ijk->ijm', cc[:, 1], corb))
        lmod = jnp.transpose(p['l_coeff'][it][:, spec_idx], (1, 0, 2))
        corb = corb * lmod

    atom_out = (_silu(density @ p['outW1'] + p['outB1']) @ p['outW2']
                + p['outB2'])[:, 0]
    sc = p['scale'].reshape(NSPEC, 2)[spec_idx]
    atom_energy = (atom_out * sc[:, 0] + sc[:, 1]) * center_factor
    energy = jax.ops.segment_sum(atom_energy, celllist, num_segments=G,
                                 indices_are_sorted=True)
    return energy


# speedup vs baseline: 1.7050x; 1.0729x over previous
import functools

import jax
import jax.numpy as jnp
import numpy as np
from jax import lax
from jax.experimental import pallas as pl
from jax.experimental.pallas import tpu as pltpu
from jax.experimental.pallas import tpu_sc as plsc

N = 10000
E = 160000
G = 8
NSPEC = 4
NWAVE = 16
PRMAXL = 3
PNORB = 9
MP_LOOP = 2
CUTOFF = 5.0
PN = 2.0
EPS = 1e-8
PIDX = (0, 1, 1, 1, 2, 2, 2, 2, 2)

EP = 163840
BLK = 2048
CH = 128

_NC = 2
_NS = 16
_NW = _NC * _NS
_PER_TILE = EP // _NW
_NCHUNK = _PER_TILE // CH
_NPAD = N
_NROWS = _NPAD // _NS
_PAD_NODE = N - 1
_ZROWS = 125


def _silu(x):
    return x * jax.nn.sigmoid(x)


def _sc_scatter_multi(vals_list, idx3d):
    nv = len(vals_list)
    Cs = [int(v.shape[1]) for v in vals_list]
    mesh = plsc.VectorSubcoreMesh(core_axis_name="c", subcore_axis_name="s")

    scratch = [pltpu.VMEM((_NCHUNK, CH), jnp.int32)]
    scratch += [pltpu.VMEM((2, CH, C), jnp.float32) for C in Cs]
    scratch += [pltpu.VMEM_SHARED((_NPAD, C), jnp.float32) for C in Cs]
    scratch += [pltpu.SemaphoreType.DMA] * (2 * nv)

    @functools.partial(
        pl.kernel, mesh=mesh,
        compiler_params=pltpu.CompilerParams(use_tc_tiling_on_sc=False),
        out_type=tuple(jax.ShapeDtypeStruct((_NC, _NPAD, C), jnp.float32)
                       for C in Cs),
        scratch_types=scratch,
    )
    def k(*refs):
        i = 0
        vals_hbm = refs[i:i + nv]; i += nv
        idx_hbm = refs[i]; i += 1
        zeros_hbm = refs[i:i + nv]; i += nv
        out_hbm = refs[i:i + nv]; i += nv
        idxv = refs[i]; i += 1
        bufs = refs[i:i + nv]; i += nv
        tabs = refs[i:i + nv]; i += nv
        sems = refs[i:i + 2 * nv]; i += 2 * nv

        c = lax.axis_index("c")
        s = lax.axis_index("s")
        wid = c * _NS + s
        base = wid * _PER_TILE
        for v in range(nv):
            for z in range(_NROWS // _ZROWS):
                pltpu.sync_copy(
                    zeros_hbm[v],
                    tabs[v].at[pl.ds(s * _NROWS + z * _ZROWS, _ZROWS), :])
        plsc.subcore_barrier()
        pltpu.sync_copy(idx_hbm.at[wid], idxv)

        def load(v, j, b):
            return pltpu.async_copy(
                vals_hbm[v].at[pl.ds(base + j * CH, CH), :],
                bufs[v].at[b], sems[2 * v + b])

        for v in range(nv):
            load(v, 0, 0)
            load(v, 1, 1)

        def step(j, b):
            for v in range(nv):
                pltpu.make_async_copy(
                    vals_hbm[v].at[pl.ds(base + j * CH, CH), :],
                    bufs[v].at[b], sems[2 * v + b]).wait()
                pltpu.sync_copy(bufs[v].at[b], tabs[v].at[idxv.at[j]],
                                add=True)

                @pl.when(j + 2 < _NCHUNK)
                def _():
                    load(v, j + 2, b)

        def outer(t, carry):
            step(2 * t, 0)
            step(2 * t + 1, 1)
            return carry

        lax.fori_loop(0, _NCHUNK // 2, outer, 0)
        plsc.subcore_barrier()
        for v in range(nv):
            pltpu.sync_copy(tabs[v].at[pl.ds(s * _NROWS, _NROWS), :],
                            out_hbm[v].at[c].at[pl.ds(s * _NROWS, _NROWS), :])

    zeros = [jnp.zeros((_ZROWS, C), jnp.float32) for C in Cs]
    parts = k(*vals_list, idx3d, *zeros)
    if not isinstance(parts, (tuple, list)):
        parts = (parts,)
    return [part[0, :N] + part[1, :N] for part in parts]


def _sc_gather2(table, idx3d_a, idx3d_b):
    C = table.shape[1]
    mesh = plsc.VectorSubcoreMesh(core_axis_name="c", subcore_axis_name="s")

    @functools.partial(
        pl.kernel, mesh=mesh,
        compiler_params=pltpu.CompilerParams(use_tc_tiling_on_sc=False),
        out_type=(jax.ShapeDtypeStruct((EP, C), jnp.float32),
                  jax.ShapeDtypeStruct((EP, C), jnp.float32)),
        scratch_types=[
            pltpu.VMEM((_NCHUNK, CH), jnp.int32),
            pltpu.VMEM((_NCHUNK, CH), jnp.int32),
            pltpu.VMEM((3, CH, C), jnp.float32),
            pltpu.VMEM((3, CH, C), jnp.float32),
            pltpu.SemaphoreType.DMA,
            pltpu.SemaphoreType.DMA,
            pltpu.SemaphoreType.DMA,
            pltpu.SemaphoreType.DMA,
            pltpu.SemaphoreType.DMA,
            pltpu.SemaphoreType.DMA,
        ],
    )
    def k(tab_hbm, ia_hbm, ib_hbm, outa_hbm, outb_hbm,
          idxa, idxb, bufa, bufb, sa0, sa1, sa2, sb0, sb1, sb2):
        c = lax.axis_index("c")
        s = lax.axis_index("s")
        wid = c * _NS + s
        base = wid * _PER_TILE
        sas = (sa0, sa1, sa2)
        sbs = (sb0, sb1, sb2)
        pltpu.sync_copy(ia_hbm.at[wid], idxa)
        pltpu.sync_copy(ib_hbm.at[wid], idxb)

        def issue(j, b):
            pltpu.async_copy(tab_hbm.at[idxa.at[j]], bufa.at[b], sas[b])
            pltpu.async_copy(tab_hbm.at[idxb.at[j]], bufb.at[b], sbs[b])

        issue(0, 0)
        issue(1, 1)
        issue(2, 2)

        def step(j, b):
            pltpu.make_async_copy(tab_hbm.at[idxa.at[j]], bufa.at[b],
                                  sas[b]).wait()
            pltpu.make_async_copy(tab_hbm.at[idxb.at[j]], bufb.at[b],
                                  sbs[b]).wait()
            pltpu.sync_copy(bufa.at[b],
                            outa_hbm.at[pl.ds(base + j * CH, CH), :])
            pltpu.sync_copy(bufb.at[b],
                            outb_hbm.at[pl.ds(base + j * CH, CH), :])

            @pl.when(j + 3 < _NCHUNK)
            def _():
                issue(j + 3, b)

        def outer(t, carry):
            step(3 * t, 0)
            step(3 * t + 1, 1)
            step(3 * t + 2, 2)
            return carry

        lax.fori_loop(0, _NCHUNK // 3, outer, 0)
        for j in range((_NCHUNK // 3) * 3, _NCHUNK):
            step(j, j % 3)

    return k(table, idx3d_a, idx3d_b)


def _edge1_body(gs_ref, gd_ref, sh_ref, cellm_ref, embt_ref, ieadt_ref,
                rdW1_ref, rdW2a_ref, rdW2bA_ref, rdW2bB_ref,
                e2W1_ref, e2W2e_ref, psA_ref, psB_ref,
                geo_ref, ead_ref, wdc_ref, worbA_ref, worbB_ref, rad_ref):
    gst = jnp.transpose(gs_ref[...])
    gdt = jnp.transpose(gd_ref[...])
    sht = sh_ref[...]
    cellm = cellm_ref[...]
    nedge = gs_ref.shape[0]

    xs, ys, zs = gst[0:1], gst[1:2], gst[2:3]
    xd, yd, zd = gdt[0:1], gdt[1:2], gdt[2:3]
    spec_s, cidx = gst[4:5], gst[5:6]
    spec_d = gdt[4:5]

    oh = [(cidx == float(g)).astype(jnp.float32) for g in range(G)]
    cmv = []
    for q in range(9):
        acc = None
        for g in range(G):
            t = oh[g] * cellm[g:g + 1, q:q + 1]
            acc = t if acc is None else acc + t
        cmv.append(acc)
    sv = [sht[0:1] * cmv[0 + kk] + sht[1:2] * cmv[3 + kk]
          + sht[2:3] * cmv[6 + kk] for kk in range(3)]

    dx = xd - xs + sv[0]
    dy = yd - ys + sv[1]
    dz = zd - zs + sv[2]
    distsq = dx * dx + dy * dy + dz * dz
    nf = (distsq > EPS).astype(jnp.float32)
    dist = jnp.sqrt(distsq + EPS)
    inv = 1.0 / dist
    ux, uy, uz = dx * inv, dy * inv, dz * inv
    s = [jnp.ones_like(ux), ux, uy, uz, ux * uy, uy * uz,
         3.0 * uz * uz - 1.0, uz * ux, ux * ux - uy * uy]
    n1 = ux * ux + uy * uy + uz * uz + EPS
    n2 = (s[4] * s[4] + s[5] * s[5] + s[6] * s[6] + s[7] * s[7]
          + s[8] * s[8] + EPS)
    f = [lax.rsqrt(jnp.ones_like(ux) + EPS),
         jnp.sqrt(3.0) * lax.rsqrt(n1),
         jnp.sqrt(5.0) * lax.rsqrt(n2)]
    sph = [s[j] * f[PIDX[j]] for j in range(PNORB)]

    nd = dist * (1.0 / CUTOFF)
    poly = 1.0 - nd * nd * ((PN + 1.0) * (PN + 2.0) / 2.0
                            - PN * (PN + 2.0) * nd
                            + PN * (PN + 1.0) / 2.0 * nd * nd)
    cut = poly * poly * nf
    pi = spec_s * float(NSPEC) + spec_d

    geo_rows = jnp.concatenate(
        sph + [nd, cut, pi, jnp.zeros((4, nedge), jnp.float32)], axis=0)
    geo = jnp.transpose(geo_rows)
    geo_ref[...] = geo

    nd_c = geo[:, 9:10]
    cut_c = geo[:, 10:11]
    pi_c = geo[:, 11:12]
    lane = lax.broadcasted_iota(
        jnp.int32, (1, NSPEC * NSPEC), 1).astype(jnp.float32)
    ohp = (pi_c == lane).astype(jnp.float32)
    embc = jnp.dot(ohp, embt_ref[...], preferred_element_type=jnp.float32)
    iead = jnp.dot(ohp, ieadt_ref[...], preferred_element_type=jnp.float32)

    smooth = iead * cut_c
    rf = jnp.sinc(nd_c * embc) * cut_c
    radial_func = jnp.concatenate([smooth[:, NWAVE:], rf], axis=1)
    h = _silu(jnp.dot(radial_func, rdW1_ref[...],
                      preferred_element_type=jnp.float32))
    wr_a = jnp.dot(h, rdW2a_ref[...], preferred_element_type=jnp.float32)
    ead = jnp.concatenate([smooth[:, :NWAVE], wr_a[:, NWAVE:]], axis=1)
    ead_ref[...] = ead
    wdc_ref[...] = jnp.concatenate(
        [wr_a[:, :NWAVE], cut_c,
         jnp.zeros((nedge, NWAVE - 1), jnp.float32)], axis=1)

    sphxA = jnp.dot(geo, psA_ref[...], preferred_element_type=jnp.float32)
    sphxB = jnp.dot(geo, psB_ref[...], preferred_element_type=jnp.float32)
    worbA_ref[...] = jnp.dot(h, rdW2bA_ref[...],
                             preferred_element_type=jnp.float32) * sphxA
    worbB_ref[...] = jnp.dot(h, rdW2bB_ref[...],
                             preferred_element_type=jnp.float32) * sphxB

    h2 = _silu(jnp.dot(ead, e2W1_ref[...],
                       preferred_element_type=jnp.float32))
    rad_ref[...] = jnp.dot(h2, e2W2e_ref[...],
                           preferred_element_type=jnp.float32)


def _edge_phase1(gs, gd, shp, cellm, embt, ieadt, rdW1, rdW2a, rdW2bA,
                 rdW2bB, e2W1, e2W2e, psA, psB):
    def eb(c):
        return pl.BlockSpec((BLK, c), lambda i: (i, 0))

    def wb(shape):
        return pl.BlockSpec(shape, lambda i: tuple(0 for _ in shape))

    outs = (
        jax.ShapeDtypeStruct((EP, NWAVE), jnp.float32),
        jax.ShapeDtypeStruct((EP, 2 * NWAVE), jnp.float32),
        jax.ShapeDtypeStruct((EP, 2 * NWAVE), jnp.float32),
        jax.ShapeDtypeStruct((EP, 5 * NWAVE), jnp.float32),
        jax.ShapeDtypeStruct((EP, 4 * NWAVE), jnp.float32),
        jax.ShapeDtypeStruct((EP, 27 * NWAVE), jnp.float32),
    )
    return pl.pallas_call(
        _edge1_body,
        grid=(EP // BLK,),
        in_specs=[eb(16), eb(16),
                  pl.BlockSpec((8, BLK), lambda i: (0, i)),
                  wb(cellm.shape), wb(embt.shape), wb(ieadt.shape),
                  wb(rdW1.shape), wb(rdW2a.shape), wb(rdW2bA.shape),
                  wb(rdW2bB.shape), wb(e2W1.shape), wb(e2W2e.shape),
                  wb(psA.shape), wb(psB.shape)],
        out_specs=(eb(NWAVE), eb(2 * NWAVE), eb(2 * NWAVE),
                   eb(5 * NWAVE), eb(4 * NWAVE), eb(27 * NWAVE)),
        out_shape=outs,
    )(gs, gd, shp, cellm, embt, ieadt, rdW1, rdW2a, rdW2bA, rdW2bB,
      e2W1, e2W2e, psA, psB)


def _edge2_body(has_ead_out, has_nworb, ead_parts, refs):
    i = 0
    eads = []
    for _ in range(ead_parts):
        eads.append(refs[i][...])
        i += 1
    geo = refs[i][...]; i += 1
    rad = refs[i][...]; i += 1
    nc0 = refs[i][...]; i += 1
    nc1 = refs[i][...]; i += 1
    if has_nworb:
        mpW1 = refs[i][...]; i += 1
        mpW2e = refs[i][...]; i += 1
    psA = refs[i][...]; i += 1
    psB = refs[i][...]; i += 1
    qmat = refs[i][...]; i += 1
    if has_ead_out:
        eW1 = refs[i][...]; i += 1
        eW2e = refs[i][...]; i += 1
    ne_ref = refs[i]; i += 1
    orbA_ref = refs[i]; i += 1
    orbB_ref = refs[i]; i += 1
    if has_nworb:
        nworbA_ref = refs[i]; i += 1
        nworbB_ref = refs[i]; i += 1
    if has_ead_out:
        radnew_ref = refs[i]; i += 1

    W = PNORB * NWAVE
    sphxA = jnp.dot(geo, psA, preferred_element_type=jnp.float32)
    sphxB = jnp.dot(geo, psB, preferred_element_type=jnp.float32)
    r0A, r0B = rad[:, 0:80], rad[:, 80:W]
    r1A, r1B = rad[:, W:W + 80], rad[:, W + 80:2 * W]
    r2A, r2B = rad[:, 2 * W:2 * W + 80], rad[:, 2 * W + 80:3 * W]
    aoA = (r0A * nc0[:, :80] + r1A * nc1[:, :80]) * sphxA
    aoB = (r0B * nc0[:, 80:] + r1B * nc1[:, 80:]) * sphxB
    ne = (jnp.dot(aoA, qmat[:80], preferred_element_type=jnp.float32)
          + jnp.dot(aoB, qmat[80:], preferred_element_type=jnp.float32))
    ne_ref[...] = ne
    orbA_ref[...] = r2A * sphxA
    orbB_ref[...] = r2B * sphxB

    if has_nworb:
        ead_cat = jnp.concatenate(eads + [ne], axis=1)
        h = _silu(jnp.dot(ead_cat, mpW1,
                          preferred_element_type=jnp.float32))
        nwA = jnp.dot(h, mpW2e[:, :80], preferred_element_type=jnp.float32)
        nwB = jnp.dot(h, mpW2e[:, 80:], preferred_element_type=jnp.float32)
        nworbA_ref[...] = nwA * sphxA
        nworbB_ref[...] = nwB * sphxB
    if has_ead_out:
        ead_cat = jnp.concatenate(eads + [ne], axis=1)
        h2 = _silu(jnp.dot(ead_cat, eW1, preferred_element_type=jnp.float32))
        radnew_ref[...] = jnp.dot(h2, eW2e,
                                  preferred_element_type=jnp.float32)


def _edge_phase2(ead_list, geo, rad, nc0, nc1, mpW=None, psA=None, psB=None,
                 qmat=None, eadW=None):
    def eb(c):
        return pl.BlockSpec((BLK, c), lambda i: (i, 0))

    def wb(shape):
        return pl.BlockSpec(shape, lambda i: tuple(0 for _ in shape))

    has_ead_out = eadW is not None
    has_nworb = mpW is not None
    n_ead = len(ead_list)
    in_specs = ([eb(e.shape[1]) for e in ead_list]
                + [eb(NWAVE), eb(27 * NWAVE),
                   eb(PNORB * NWAVE), eb(PNORB * NWAVE)])
    args = list(ead_list) + [geo, rad, nc0, nc1]
    if has_nworb:
        in_specs += [wb(mpW[0].shape), wb(mpW[1].shape)]
        args += [mpW[0], mpW[1]]
    in_specs += [wb(psA.shape), wb(psB.shape), wb(qmat.shape)]
    args += [psA, psB, qmat]
    if has_ead_out:
        in_specs += [wb(eadW[0].shape), wb(eadW[1].shape)]
        args += [eadW[0], eadW[1]]
    outs = [jax.ShapeDtypeStruct((EP, NWAVE), jnp.float32),
            jax.ShapeDtypeStruct((EP, 5 * NWAVE), jnp.float32),
            jax.ShapeDtypeStruct((EP, 4 * NWAVE), jnp.float32)]
    out_specs = [eb(NWAVE), eb(5 * NWAVE), eb(4 * NWAVE)]
    if has_nworb:
        outs += [jax.ShapeDtypeStruct((EP, 5 * NWAVE), jnp.float32),
                 jax.ShapeDtypeStruct((EP, 4 * NWAVE), jnp.float32)]
        out_specs += [eb(5 * NWAVE), eb(4 * NWAVE)]
    if has_ead_out:
        outs.append(jax.ShapeDtypeStruct((EP, 27 * NWAVE), jnp.float32))
        out_specs.append(eb(27 * NWAVE))

    def body(*refs):
        _edge2_body(has_ead_out, has_nworb, n_ead, refs)

    return pl.pallas_call(
        body,
        grid=(EP // BLK,),
        in_specs=in_specs,
        out_specs=tuple(out_specs),
        out_shape=tuple(outs),
    )(*args)


BN = 2000


def _neb(c):
    return pl.BlockSpec((BN, c), lambda i: (i, 0))


def _nwb(shape):
    return pl.BlockSpec(shape, lambda i: tuple(0 for _ in shape))


def _spec_onehot(aux):
    lane = lax.broadcasted_iota(jnp.int32, (1, NSPEC), 1).astype(jnp.float32)
    return (aux[:, 0:1] == lane).astype(jnp.float32)


def _sel_matmul(x, oh, bd_ref):
    acc = None
    for s_ in range(NSPEC):
        t = oh[:, s_:s_ + 1] * jnp.dot(x, bd_ref[s_],
                                       preferred_element_type=jnp.float32)
        acc = t if acc is None else acc + t
    return acc


def _nodeA_body(wdc_ref, craw_ref, aux_ref, bdA_ref, ncf_ref):
    wdc = wdc_ref[...]
    ave = wdc[:, NWAVE:NWAVE + 1] + EPS
    cdiv = craw_ref[...] / ave
    oh = _spec_onehot(aux_ref[...])
    ncf_ref[...] = _sel_matmul(cdiv, oh, bdA_ref)


def _node_a(wdc_n, corb_raw, aux, bdA):
    return pl.pallas_call(
        _nodeA_body,
        grid=(N // BN,),
        in_specs=[_neb(2 * NWAVE), _neb(PNORB * NWAVE), _neb(8),
                  _nwb(bdA.shape)],
        out_specs=_neb(PNORB * NWAVE),
        out_shape=jax.ShapeDtypeStruct((N, PNORB * NWAVE), jnp.float32),
    )(wdc_n, corb_raw, aux, bdA)


def _nodeB_body(wdc_ref, ncf_ref, so_ref, sn_ref, aux_ref, bd0_ref, bd1_ref,
                ltab_ref, qo_ref, d1_ref, ncfn_ref):
    wdc = wdc_ref[...]
    ave = wdc[:, NWAVE:NWAVE + 1] + EPS
    ncf = ncf_ref[...]
    so = so_ref[...]
    d1_ref[...] = jnp.dot(so * ncf, qo_ref[...],
                          preferred_element_type=jnp.float32)
    oh = _spec_onehot(aux_ref[...])
    t0 = _sel_matmul(sn_ref[...] / ave, oh, bd0_ref)
    t1 = _sel_matmul(ncf, oh, bd1_ref)
    lm = None
    for s_ in range(NSPEC):
        t = oh[:, s_:s_ + 1] * ltab_ref[s_:s_ + 1, :]
        lm = t if lm is None else lm + t
    ncfn_ref[...] = (t0 + t1) * lm


def _node_b(wdc_n, ncf, sum_orb, sum_new, aux, bd0, bd1, ltab, qones):
    return pl.pallas_call(
        _nodeB_body,
        grid=(N // BN,),
        in_specs=[_neb(2 * NWAVE), _neb(PNORB * NWAVE),
                  _neb(PNORB * NWAVE), _neb(PNORB * NWAVE), _neb(8),
                  _nwb(bd0.shape), _nwb(bd1.shape), _nwb(ltab.shape),
                  _nwb(qones.shape)],
        out_specs=(_neb(NWAVE), _neb(PNORB * NWAVE)),
        out_shape=(jax.ShapeDtypeStruct((N, NWAVE), jnp.float32),
                   jax.ShapeDtypeStruct((N, PNORB * NWAVE), jnp.float32)),
    )(wdc_n, ncf, sum_orb, sum_new, aux, bd0, bd1, ltab, qones)


def _nodeC_body(wdc_ref, d1_ref, ncf_ref, so_ref, aux_ref, qo_ref,
                oW1_ref, oB1_ref, oW2_ref, scm_ref, out_ref, acc_ref):
    i = pl.program_id(0)

    @pl.when(i == 0)
    def _():
        acc_ref[...] = jnp.zeros_like(acc_ref)

    wdc = wdc_ref[...]
    d2 = jnp.dot(so_ref[...] * ncf_ref[...], qo_ref[...],
                 preferred_element_type=jnp.float32)
    density = jnp.concatenate([wdc[:, :NWAVE], d1_ref[...], d2], axis=1)
    h = _silu(jnp.dot(density, oW1_ref[...],
                      preferred_element_type=jnp.float32) + oB1_ref[...])
    atom = jnp.dot(h, oW2_ref[...], preferred_element_type=jnp.float32)
    aux = aux_ref[...]
    oh = _spec_onehot(aux)
    scv = jnp.dot(oh, scm_ref[...], preferred_element_type=jnp.float32)
    ae = (atom[:, 0:1] * scv[:, 0:1] + scv[:, 1:2]) * aux[:, 1:2]
    lane = lax.broadcasted_iota(jnp.int32, (1, G), 1).astype(jnp.float32)
    ohg = (aux[:, 2:3] == lane).astype(jnp.float32)
    acc_ref[...] += jnp.sum(ohg * ae, axis=0, keepdims=True)

    @pl.when(i == pl.num_programs(0) - 1)
    def _():
        out_ref[...] = acc_ref[...]


def _node_c(wdc_n, d1, ncf, sum_orb, aux, qones, oW1, oB1, oW2pad, scm):
    return pl.pallas_call(
        _nodeC_body,
        grid=(N // BN,),
        in_specs=[_neb(2 * NWAVE), _neb(NWAVE), _neb(PNORB * NWAVE),
                  _neb(PNORB * NWAVE), _neb(8), _nwb(qones.shape),
                  _nwb(oW1.shape), _nwb(oB1.shape), _nwb(oW2pad.shape),
                  _nwb(scm.shape)],
        out_specs=pl.BlockSpec((1, G), lambda i: (0, 0)),
        out_shape=jax.ShapeDtypeStruct((1, G), jnp.float32),
        scratch_shapes=[pltpu.VMEM((1, G), jnp.float32)],
    )(wdc_n, d1, ncf, sum_orb, aux, qones, oW1, oB1, oW2pad, scm)


def _expand_blocks(W, blocks):
    H = W.shape[0]
    Wb = W.reshape(H, -1, NWAVE)
    return Wb[:, list(blocks)].reshape(H, len(blocks) * NWAVE)


def kernel(cart, cell, disp_cell, neighlist, celllist, shiftimage,
           center_factor, species, params):
    p = params
    f32 = jnp.float32
    com_spec = jnp.array([[float(i), float(j)] for i in range(NSPEC)
                          for j in range(NSPEC)], dtype=f32)

    symm_cell = (disp_cell + jnp.transpose(disp_cell, (0, 2, 1))) / 2.0
    cell = cell + jnp.einsum('ijk,ikm->ijm', cell, symm_cell)
    symm_cell_n = symm_cell[celllist]
    cart = cart + jnp.einsum('ij,ijk->ik', cart, symm_cell_n)
    cellm = jnp.concatenate(
        [cell.reshape(G, 9), jnp.zeros((G, 7), f32)], axis=1)

    pad_idx = jnp.full((EP - E,), _PAD_NODE, jnp.int32)
    idx0 = jnp.concatenate([neighlist[0], pad_idx]).reshape(_NW, _NCHUNK, CH)
    idx1 = jnp.concatenate([neighlist[1], pad_idx]).reshape(_NW, _NCHUNK, CH)
    spec_idx = species

    node_tab = jnp.concatenate(
        [cart, jnp.zeros((N, 1), f32), spec_idx[:, None].astype(f32),
         celllist[:, None].astype(f32), jnp.zeros((N, 10), f32)], axis=1)
    gs, gd = _sc_gather2(node_tab, idx0, idx1)

    shp = jnp.concatenate(
        [shiftimage, jnp.zeros((3, EP - E), f32)], axis=1)
    shp = jnp.concatenate([shp, jnp.zeros((5, EP), f32)], axis=0)

    pair_spec = _silu(com_spec @ p['ncW1'] + p['ncB1']) @ p['ncW2'] + p['ncB2']
    embt = (_silu(pair_spec @ p['nnW1'] + p['nnB1']) @ p['nnW2']
            + p['nnB2'])
    ieadt = _silu(pair_spec @ p['rwW1']) @ p['rwW2']

    psA_np = np.zeros((NWAVE, 5 * NWAVE), np.float32)
    psB_np = np.zeros((NWAVE, 4 * NWAVE), np.float32)
    for j in range(5):
        psA_np[j, j * NWAVE:(j + 1) * NWAVE] = 1.0
    for j in range(5, PNORB):
        psB_np[j, (j - 5) * NWAVE:(j - 4) * NWAVE] = 1.0
    psA = jnp.asarray(psA_np)
    psB = jnp.asarray(psB_np)
    q_np = np.zeros((PNORB * NWAVE, NWAVE), np.float32)
    for j in range(PNORB):
        for m in range(NWAVE):
            q_np[j * NWAVE + m, m] = 1.0 / np.sqrt(2.0)
    qmat = jnp.asarray(q_np)

    rdW2 = p['rdW2']
    rdW2a = rdW2[:, 3 * NWAVE:5 * NWAVE]
    rdW2bA = _expand_blocks(rdW2, PIDX[:5])
    rdW2bB = _expand_blocks(rdW2, PIDX[5:])
    rxblocks = [r * PRMAXL + PIDX[j] for r in range(3) for j in range(PNORB)]
    e2W2e = _expand_blocks(p['ead2W2'], rxblocks)
    mpW2e = [_expand_blocks(p['mp0W2'], PIDX),
             _expand_blocks(p['mp1W2'], PIDX)]
    ead0W2e = _expand_blocks(p['ead0W2'], rxblocks)

    geo, ead0, wdc, worbA, worbB, rad = _edge_phase1(
        gs, gd, shp, cellm, embt, ieadt, p['rdW1'], rdW2a, rdW2bA, rdW2bB,
        p['ead2W1'], e2W2e, psA, psB)

    wdc_n = _sc_scatter_multi([wdc], idx0)[0]
    corb_raw = jnp.concatenate(
        [_sc_scatter_multi([worbA], idx0)[0],
         _sc_scatter_multi([worbB], idx0)[0]], axis=1)

    rt3 = np.sqrt(float(PRMAXL))
    eye9 = jnp.eye(PNORB, dtype=f32)
    bdA = jnp.stack(
        [jnp.kron(eye9, p['spec_coeff'][s_] / np.sqrt(float(NWAVE))) / rt3
         for s_ in range(NSPEC)])
    cc0 = p['contract_coeff'][0]
    bd0 = jnp.stack(
        [jnp.kron(eye9, cc0[s_, 0] / np.sqrt(float(NWAVE))) for s_ in
         range(NSPEC)])
    bd1 = jnp.stack(
        [jnp.kron(eye9, cc0[s_, 1] * rt3) for s_ in range(NSPEC)])
    ltab = (jnp.transpose(p['l_coeff'][0], (1, 0, 2))
            .reshape(NSPEC, PNORB * NWAVE) / rt3)
    qo_np = np.zeros((PNORB * NWAVE, NWAVE), np.float32)
    for j in range(PNORB):
        for m in range(NWAVE):
            qo_np[j * NWAVE + m, m] = 1.0
    qones = jnp.asarray(qo_np)
    aux = jnp.concatenate(
        [spec_idx[:, None].astype(f32), center_factor[:, None],
         celllist[:, None].astype(f32), jnp.zeros((N, 5), f32)], axis=1)
    scl = p['scale'].reshape(NSPEC, 2)
    scm = jnp.stack([scl[:, 0], p['outB2'][0] * scl[:, 0] + scl[:, 1]],
                    axis=1)
    oW2pad = jnp.concatenate(
        [p['outW2'], jnp.zeros((p['outW2'].shape[0], G - 1), f32)], axis=1)
    oB1 = p['outB1'][None, :]

    ncf = _node_a(wdc_n, corb_raw, aux, bdA)

    nc0, nc1 = _sc_gather2(ncf, idx0, idx1)
    res = _edge_phase2([ead0], geo, rad, nc0, nc1,
                       (p['mp0W1'], mpW2e[0]), psA, psB, qmat,
                       (p['ead0W1'], ead0W2e))
    ne1, orbA0, orbB0, nworbA, nworbB, rad = res
    sum_orb0 = jnp.concatenate(
        [_sc_scatter_multi([orbA0], idx0)[0],
         _sc_scatter_multi([orbB0], idx0)[0]], axis=1)
    sum_new = jnp.concatenate(
        [_sc_scatter_multi([nworbA], idx0)[0],
         _sc_scatter_multi([nworbB], idx0)[0]], axis=1)
    d1, ncf = _node_b(wdc_n, ncf, sum_orb0, sum_new, aux, bd0, bd1, ltab,
                      qones)

    nc0, nc1 = _sc_gather2(ncf, idx0, idx1)
    ne2, orbA1, orbB1 = _edge_phase2([ead0, ne1], geo, rad, nc0, nc1,
                                     None, psA, psB, qmat, None)
    sum_orb1 = jnp.concatenate(
        [_sc_scatter_multi([orbA1], idx0)[0],
         _sc_scatter_multi([orbB1], idx0)[0]], axis=1)

    energy = _node_c(wdc_n, d1, ncf, sum_orb1, aux, qones,
                     p['outW1'], oB1, oW2pad, scm)
    return energy.reshape(G)


# speedup vs baseline: 1.7091x; 1.0024x over previous
import functools

import jax
import jax.numpy as jnp
import numpy as np
from jax import lax
from jax.experimental import pallas as pl
from jax.experimental.pallas import tpu as pltpu
from jax.experimental.pallas import tpu_sc as plsc

N = 10000
E = 160000
G = 8
NSPEC = 4
NWAVE = 16
PRMAXL = 3
PNORB = 9
MP_LOOP = 2
CUTOFF = 5.0
PN = 2.0
EPS = 1e-8
PIDX = (0, 1, 1, 1, 2, 2, 2, 2, 2)

EP = 163840
BLK = 2048
CH = 128

_NC = 2
_NS = 16
_NW = _NC * _NS
_PER_TILE = EP // _NW
_NCHUNK = _PER_TILE // CH
_NPAD = N
_NROWS = _NPAD // _NS
_PAD_NODE = N - 1
_ZROWS = 125


def _silu(x):
    return x * jax.nn.sigmoid(x)


def _sc_scatter_multi(vals_list, idx3d):
    nv = len(vals_list)
    Cs = [int(v.shape[1]) for v in vals_list]
    mesh = plsc.VectorSubcoreMesh(core_axis_name="c", subcore_axis_name="s")

    scratch = [pltpu.VMEM((_NCHUNK, CH), jnp.int32)]
    scratch += [pltpu.VMEM((2, CH, C), jnp.float32) for C in Cs]
    scratch += [pltpu.VMEM_SHARED((_NPAD, C), jnp.float32) for C in Cs]
    scratch += [pltpu.SemaphoreType.DMA] * (2 * nv)

    @functools.partial(
        pl.kernel, mesh=mesh,
        compiler_params=pltpu.CompilerParams(use_tc_tiling_on_sc=False),
        out_type=tuple(jax.ShapeDtypeStruct((_NC, _NPAD, C), jnp.float32)
                       for C in Cs),
        scratch_types=scratch,
    )
    def k(*refs):
        i = 0
        vals_hbm = refs[i:i + nv]; i += nv
        idx_hbm = refs[i]; i += 1
        zeros_hbm = refs[i:i + nv]; i += nv
        out_hbm = refs[i:i + nv]; i += nv
        idxv = refs[i]; i += 1
        bufs = refs[i:i + nv]; i += nv
        tabs = refs[i:i + nv]; i += nv
        sems = refs[i:i + 2 * nv]; i += 2 * nv

        c = lax.axis_index("c")
        s = lax.axis_index("s")
        wid = c * _NS + s
        base = wid * _PER_TILE
        for v in range(nv):
            for z in range(_NROWS // _ZROWS):
                pltpu.sync_copy(
                    zeros_hbm[v],
                    tabs[v].at[pl.ds(s * _NROWS + z * _ZROWS, _ZROWS), :])
        plsc.subcore_barrier()
        pltpu.sync_copy(idx_hbm.at[wid], idxv)

        def load(v, j, b):
            return pltpu.async_copy(
                vals_hbm[v].at[pl.ds(base + j * CH, CH), :],
                bufs[v].at[b], sems[2 * v + b])

        for v in range(nv):
            load(v, 0, 0)
            load(v, 1, 1)

        def step(j, b):
            for v in range(nv):
                pltpu.make_async_copy(
                    vals_hbm[v].at[pl.ds(base + j * CH, CH), :],
                    bufs[v].at[b], sems[2 * v + b]).wait()
                pltpu.sync_copy(bufs[v].at[b], tabs[v].at[idxv.at[j]],
                                add=True)

                @pl.when(j + 2 < _NCHUNK)
                def _():
                    load(v, j + 2, b)

        def outer(t, carry):
            step(2 * t, 0)
            step(2 * t + 1, 1)
            return carry

        lax.fori_loop(0, _NCHUNK // 2, outer, 0)
        plsc.subcore_barrier()
        for v in range(nv):
            pltpu.sync_copy(tabs[v].at[pl.ds(s * _NROWS, _NROWS), :],
                            out_hbm[v].at[c].at[pl.ds(s * _NROWS, _NROWS), :])

    zeros = [jnp.zeros((_ZROWS, C), jnp.float32) for C in Cs]
    parts = k(*vals_list, idx3d, *zeros)
    if not isinstance(parts, (tuple, list)):
        parts = (parts,)
    return [part[0, :N] + part[1, :N] for part in parts]


def _sc_gather2(table, idx3d_a, idx3d_b):
    C = table.shape[1]
    mesh = plsc.VectorSubcoreMesh(core_axis_name="c", subcore_axis_name="s")

    @functools.partial(
        pl.kernel, mesh=mesh,
        compiler_params=pltpu.CompilerParams(use_tc_tiling_on_sc=False),
        out_type=(jax.ShapeDtypeStruct((EP, C), jnp.float32),
                  jax.ShapeDtypeStruct((EP, C), jnp.float32)),
        scratch_types=[
            pltpu.VMEM((_NCHUNK, CH), jnp.int32),
            pltpu.VMEM((_NCHUNK, CH), jnp.int32),
            pltpu.VMEM((3, CH, C), jnp.float32),
            pltpu.VMEM((3, CH, C), jnp.float32),
            pltpu.SemaphoreType.DMA,
            pltpu.SemaphoreType.DMA,
            pltpu.SemaphoreType.DMA,
            pltpu.SemaphoreType.DMA,
            pltpu.SemaphoreType.DMA,
            pltpu.SemaphoreType.DMA,
        ],
    )
    def k(tab_hbm, ia_hbm, ib_hbm, outa_hbm, outb_hbm,
          idxa, idxb, bufa, bufb, sa0, sa1, sa2, sb0, sb1, sb2):
        c = lax.axis_index("c")
        s = lax.axis_index("s")
        wid = c * _NS + s
        base = wid * _PER_TILE
        sas = (sa0, sa1, sa2)
        sbs = (sb0, sb1, sb2)
        pltpu.sync_copy(ia_hbm.at[wid], idxa)
        pltpu.sync_copy(ib_hbm.at[wid], idxb)

        def issue(j, b):
            pltpu.async_copy(tab_hbm.at[idxa.at[j]], bufa.at[b], sas[b])
            pltpu.async_copy(tab_hbm.at[idxb.at[j]], bufb.at[b], sbs[b])

        issue(0, 0)
        issue(1, 1)
        issue(2, 2)

        def step(j, b):
            pltpu.make_async_copy(tab_hbm.at[idxa.at[j]], bufa.at[b],
                                  sas[b]).wait()
            pltpu.make_async_copy(tab_hbm.at[idxb.at[j]], bufb.at[b],
                                  sbs[b]).wait()
            pltpu.sync_copy(bufa.at[b],
                            outa_hbm.at[pl.ds(base + j * CH, CH), :])
            pltpu.sync_copy(bufb.at[b],
                            outb_hbm.at[pl.ds(base + j * CH, CH), :])

            @pl.when(j + 3 < _NCHUNK)
            def _():
                issue(j + 3, b)

        def outer(t, carry):
            step(3 * t, 0)
            step(3 * t + 1, 1)
            step(3 * t + 2, 2)
            return carry

        lax.fori_loop(0, _NCHUNK // 3, outer, 0)
        for j in range((_NCHUNK // 3) * 3, _NCHUNK):
            step(j, j % 3)

    return k(table, idx3d_a, idx3d_b)


def _edge1_body(gs_ref, gd_ref, sh_ref, cellm_ref, embt_ref, ieadt_ref,
                rdW1_ref, rdW2a_ref, rdW2bA_ref, rdW2bB_ref,
                e2W1_ref, e2W2e_ref, psA_ref, psB_ref,
                geo_ref, ead_ref, wdc_ref, worbA_ref, worbB_ref, rad_ref):
    gst = jnp.transpose(gs_ref[...])
    gdt = jnp.transpose(gd_ref[...])
    sht = sh_ref[...]
    cellm = cellm_ref[...]
    nedge = gs_ref.shape[0]

    xs, ys, zs = gst[0:1], gst[1:2], gst[2:3]
    xd, yd, zd = gdt[0:1], gdt[1:2], gdt[2:3]
    spec_s, cidx = gst[4:5], gst[5:6]
    spec_d = gdt[4:5]

    oh = [(cidx == float(g)).astype(jnp.float32) for g in range(G)]
    cmv = []
    for q in range(9):
        acc = None
        for g in range(G):
            t = oh[g] * cellm[g:g + 1, q:q + 1]
            acc = t if acc is None else acc + t
        cmv.append(acc)
    sv = [sht[0:1] * cmv[0 + kk] + sht[1:2] * cmv[3 + kk]
          + sht[2:3] * cmv[6 + kk] for kk in range(3)]

    dx = xd - xs + sv[0]
    dy = yd - ys + sv[1]
    dz = zd - zs + sv[2]
    distsq = dx * dx + dy * dy + dz * dz
    nf = (distsq > EPS).astype(jnp.float32)
    dist = jnp.sqrt(distsq + EPS)
    inv = 1.0 / dist
    ux, uy, uz = dx * inv, dy * inv, dz * inv
    s = [jnp.ones_like(ux), ux, uy, uz, ux * uy, uy * uz,
         3.0 * uz * uz - 1.0, uz * ux, ux * ux - uy * uy]
    n1 = ux * ux + uy * uy + uz * uz + EPS
    n2 = (s[4] * s[4] + s[5] * s[5] + s[6] * s[6] + s[7] * s[7]
          + s[8] * s[8] + EPS)
    f = [1.0 / jnp.sqrt(jnp.ones_like(ux) + EPS),
         jnp.sqrt(3.0) / jnp.sqrt(n1),
         jnp.sqrt(5.0) / jnp.sqrt(n2)]
    sph = [s[j] * f[PIDX[j]] for j in range(PNORB)]

    nd = dist * (1.0 / CUTOFF)
    poly = 1.0 - nd * nd * ((PN + 1.0) * (PN + 2.0) / 2.0
                            - PN * (PN + 2.0) * nd
                            + PN * (PN + 1.0) / 2.0 * nd * nd)
    cut = poly * poly * nf
    pi = spec_s * float(NSPEC) + spec_d

    geo_rows = jnp.concatenate(
        sph + [nd, cut, pi, jnp.zeros((4, nedge), jnp.float32)], axis=0)
    geo = jnp.transpose(geo_rows)
    geo_ref[...] = geo

    nd_c = geo[:, 9:10]
    cut_c = geo[:, 10:11]
    pi_c = geo[:, 11:12]
    lane = lax.broadcasted_iota(
        jnp.int32, (1, NSPEC * NSPEC), 1).astype(jnp.float32)
    ohp = (pi_c == lane).astype(jnp.float32)
    embc = jnp.dot(ohp, embt_ref[...], preferred_element_type=jnp.float32)
    iead = jnp.dot(ohp, ieadt_ref[...], preferred_element_type=jnp.float32)

    smooth = iead * cut_c
    rf = jnp.sinc(nd_c * embc) * cut_c
    radial_func = jnp.concatenate([smooth[:, NWAVE:], rf], axis=1)
    h = _silu(jnp.dot(radial_func, rdW1_ref[...],
                      preferred_element_type=jnp.float32))
    wr_a = jnp.dot(h, rdW2a_ref[...], preferred_element_type=jnp.float32)
    ead = jnp.concatenate([smooth[:, :NWAVE], wr_a[:, NWAVE:]], axis=1)
    ead_ref[...] = ead
    wdc_ref[...] = jnp.concatenate(
        [wr_a[:, :NWAVE], cut_c,
         jnp.zeros((nedge, NWAVE - 1), jnp.float32)], axis=1)

    sphxA = jnp.dot(geo, psA_ref[...], preferred_element_type=jnp.float32)
    sphxB = jnp.dot(geo, psB_ref[...], preferred_element_type=jnp.float32)
    worbA_ref[...] = jnp.dot(h, rdW2bA_ref[...],
                             preferred_element_type=jnp.float32) * sphxA
    worbB_ref[...] = jnp.dot(h, rdW2bB_ref[...],
                             preferred_element_type=jnp.float32) * sphxB

    h2 = _silu(jnp.dot(ead, e2W1_ref[...],
                       preferred_element_type=jnp.float32))
    rad_ref[...] = jnp.dot(h2, e2W2e_ref[...],
                           preferred_element_type=jnp.float32)


def _edge_phase1(gs, gd, shp, cellm, embt, ieadt, rdW1, rdW2a, rdW2bA,
                 rdW2bB, e2W1, e2W2e, psA, psB):
    def eb(c):
        return pl.BlockSpec((BLK, c), lambda i: (i, 0))

    def wb(shape):
        return pl.BlockSpec(shape, lambda i: tuple(0 for _ in shape))

    outs = (
        jax.ShapeDtypeStruct((EP, NWAVE), jnp.float32),
        jax.ShapeDtypeStruct((EP, 2 * NWAVE), jnp.float32),
        jax.ShapeDtypeStruct((EP, 2 * NWAVE), jnp.float32),
        jax.ShapeDtypeStruct((EP, 5 * NWAVE), jnp.float32),
        jax.ShapeDtypeStruct((EP, 4 * NWAVE), jnp.float32),
        jax.ShapeDtypeStruct((EP, 27 * NWAVE), jnp.float32),
    )
    return pl.pallas_call(
        _edge1_body,
        grid=(EP // BLK,),
        in_specs=[eb(16), eb(16),
                  pl.BlockSpec((8, BLK), lambda i: (0, i)),
                  wb(cellm.shape), wb(embt.shape), wb(ieadt.shape),
                  wb(rdW1.shape), wb(rdW2a.shape), wb(rdW2bA.shape),
                  wb(rdW2bB.shape), wb(e2W1.shape), wb(e2W2e.shape),
                  wb(psA.shape), wb(psB.shape)],
        out_specs=(eb(NWAVE), eb(2 * NWAVE), eb(2 * NWAVE),
                   eb(5 * NWAVE), eb(4 * NWAVE), eb(27 * NWAVE)),
        out_shape=outs,
    )(gs, gd, shp, cellm, embt, ieadt, rdW1, rdW2a, rdW2bA, rdW2bB,
      e2W1, e2W2e, psA, psB)


def _edge2_body(has_ead_out, has_nworb, ead_parts, refs):
    i = 0
    eads = []
    for _ in range(ead_parts):
        eads.append(refs[i][...])
        i += 1
    geo = refs[i][...]; i += 1
    rad = refs[i][...]; i += 1
    nc0 = refs[i][...]; i += 1
    nc1 = refs[i][...]; i += 1
    if has_nworb:
        mpW1 = refs[i][...]; i += 1
        mpW2e = refs[i][...]; i += 1
    psA = refs[i][...]; i += 1
    psB = refs[i][...]; i += 1
    qmat = refs[i][...]; i += 1
    if has_ead_out:
        eW1 = refs[i][...]; i += 1
        eW2e = refs[i][...]; i += 1
    ne_ref = refs[i]; i += 1
    orbA_ref = refs[i]; i += 1
    orbB_ref = refs[i]; i += 1
    if has_nworb:
        nworbA_ref = refs[i]; i += 1
        nworbB_ref = refs[i]; i += 1
    if has_ead_out:
        radnew_ref = refs[i]; i += 1

    W = PNORB * NWAVE
    sphxA = jnp.dot(geo, psA, preferred_element_type=jnp.float32)
    sphxB = jnp.dot(geo, psB, preferred_element_type=jnp.float32)
    r0A, r0B = rad[:, 0:80], rad[:, 80:W]
    r1A, r1B = rad[:, W:W + 80], rad[:, W + 80:2 * W]
    r2A, r2B = rad[:, 2 * W:2 * W + 80], rad[:, 2 * W + 80:3 * W]
    aoA = (r0A * nc0[:, :80] + r1A * nc1[:, :80]) * sphxA
    aoB = (r0B * nc0[:, 80:] + r1B * nc1[:, 80:]) * sphxB
    ne = (jnp.dot(aoA, qmat[:80], preferred_element_type=jnp.float32)
          + jnp.dot(aoB, qmat[80:], preferred_element_type=jnp.float32))
    ne_ref[...] = ne
    orbA_ref[...] = r2A * sphxA
    orbB_ref[...] = r2B * sphxB

    if has_nworb:
        ead_cat = jnp.concatenate(eads + [ne], axis=1)
        h = _silu(jnp.dot(ead_cat, mpW1,
                          preferred_element_type=jnp.float32))
        nwA = jnp.dot(h, mpW2e[:, :80], preferred_element_type=jnp.float32)
        nwB = jnp.dot(h, mpW2e[:, 80:], preferred_element_type=jnp.float32)
        nworbA_ref[...] = nwA * sphxA
        nworbB_ref[...] = nwB * sphxB
    if has_ead_out:
        ead_cat = jnp.concatenate(eads + [ne], axis=1)
        h2 = _silu(jnp.dot(ead_cat, eW1, preferred_element_type=jnp.float32))
        radnew_ref[...] = jnp.dot(h2, eW2e,
                                  preferred_element_type=jnp.float32)


def _edge_phase2(ead_list, geo, rad, nc0, nc1, mpW=None, psA=None, psB=None,
                 qmat=None, eadW=None):
    def eb(c):
        return pl.BlockSpec((BLK, c), lambda i: (i, 0))

    def wb(shape):
        return pl.BlockSpec(shape, lambda i: tuple(0 for _ in shape))

    has_ead_out = eadW is not None
    has_nworb = mpW is not None
    n_ead = len(ead_list)
    in_specs = ([eb(e.shape[1]) for e in ead_list]
                + [eb(NWAVE), eb(27 * NWAVE),
                   eb(PNORB * NWAVE), eb(PNORB * NWAVE)])
    args = list(ead_list) + [geo, rad, nc0, nc1]
    if has_nworb:
        in_specs += [wb(mpW[0].shape), wb(mpW[1].shape)]
        args += [mpW[0], mpW[1]]
    in_specs += [wb(psA.shape), wb(psB.shape), wb(qmat.shape)]
    args += [psA, psB, qmat]
    if has_ead_out:
        in_specs += [wb(eadW[0].shape), wb(eadW[1].shape)]
        args += [eadW[0], eadW[1]]
    outs = [jax.ShapeDtypeStruct((EP, NWAVE), jnp.float32),
            jax.ShapeDtypeStruct((EP, 5 * NWAVE), jnp.float32),
            jax.ShapeDtypeStruct((EP, 4 * NWAVE), jnp.float32)]
    out_specs = [eb(NWAVE), eb(5 * NWAVE), eb(4 * NWAVE)]
    if has_nworb:
        outs += [jax.ShapeDtypeStruct((EP, 5 * NWAVE), jnp.float32),
                 jax.ShapeDtypeStruct((EP, 4 * NWAVE), jnp.float32)]
        out_specs += [eb(5 * NWAVE), eb(4 * NWAVE)]
    if has_ead_out:
        outs.append(jax.ShapeDtypeStruct((EP, 27 * NWAVE), jnp.float32))
        out_specs.append(eb(27 * NWAVE))

    def body(*refs):
        _edge2_body(has_ead_out, has_nworb, n_ead, refs)

    return pl.pallas_call(
        body,
        grid=(EP // BLK,),
        in_specs=in_specs,
        out_specs=tuple(out_specs),
        out_shape=tuple(outs),
    )(*args)


BN = 2000


def _neb(c):
    return pl.BlockSpec((BN, c), lambda i: (i, 0))


def _nwb(shape):
    return pl.BlockSpec(shape, lambda i: tuple(0 for _ in shape))


def _spec_onehot(aux):
    lane = lax.broadcasted_iota(jnp.int32, (1, NSPEC), 1).astype(jnp.float32)
    return (aux[:, 0:1] == lane).astype(jnp.float32)


def _sel_matmul(x, oh, bd_ref):
    acc = None
    for s_ in range(NSPEC):
        t = oh[:, s_:s_ + 1] * jnp.dot(x, bd_ref[s_],
                                       preferred_element_type=jnp.float32)
        acc = t if acc is None else acc + t
    return acc


def _nodeA_body(wdc_ref, craw_ref, aux_ref, bdA_ref, ncf_ref):
    wdc = wdc_ref[...]
    ave = wdc[:, NWAVE:NWAVE + 1] + EPS
    cdiv = craw_ref[...] / ave
    oh = _spec_onehot(aux_ref[...])
    ncf_ref[...] = _sel_matmul(cdiv, oh, bdA_ref)


def _node_a(wdc_n, corb_raw, aux, bdA):
    return pl.pallas_call(
        _nodeA_body,
        grid=(N // BN,),
        in_specs=[_neb(2 * NWAVE), _neb(PNORB * NWAVE), _neb(8),
                  _nwb(bdA.shape)],
        out_specs=_neb(PNORB * NWAVE),
        out_shape=jax.ShapeDtypeStruct((N, PNORB * NWAVE), jnp.float32),
    )(wdc_n, corb_raw, aux, bdA)


def _nodeB_body(wdc_ref, ncf_ref, so_ref, sn_ref, aux_ref, bd0_ref, bd1_ref,
                ltab_ref, qo_ref, d1_ref, ncfn_ref):
    wdc = wdc_ref[...]
    ave = wdc[:, NWAVE:NWAVE + 1] + EPS
    ncf = ncf_ref[...]
    so = so_ref[...]
    d1_ref[...] = jnp.dot(so * ncf, qo_ref[...],
                          preferred_element_type=jnp.float32)
    oh = _spec_onehot(aux_ref[...])
    t0 = _sel_matmul(sn_ref[...] / ave, oh, bd0_ref)
    t1 = _sel_matmul(ncf, oh, bd1_ref)
    lm = None
    for s_ in range(NSPEC):
        t = oh[:, s_:s_ + 1] * ltab_ref[s_:s_ + 1, :]
        lm = t if lm is None else lm + t
    ncfn_ref[...] = (t0 + t1) * lm


def _node_b(wdc_n, ncf, sum_orb, sum_new, aux, bd0, bd1, ltab, qones):
    return pl.pallas_call(
        _nodeB_body,
        grid=(N // BN,),
        in_specs=[_neb(2 * NWAVE), _neb(PNORB * NWAVE),
                  _neb(PNORB * NWAVE), _neb(PNORB * NWAVE), _neb(8),
                  _nwb(bd0.shape), _nwb(bd1.shape), _nwb(ltab.shape),
                  _nwb(qones.shape)],
        out_specs=(_neb(NWAVE), _neb(PNORB * NWAVE)),
        out_shape=(jax.ShapeDtypeStruct((N, NWAVE), jnp.float32),
                   jax.ShapeDtypeStruct((N, PNORB * NWAVE), jnp.float32)),
    )(wdc_n, ncf, sum_orb, sum_new, aux, bd0, bd1, ltab, qones)


def _nodeC_body(wdc_ref, d1_ref, ncf_ref, so_ref, aux_ref, qo_ref,
                oW1_ref, oB1_ref, oW2_ref, scm_ref, out_ref, acc_ref):
    i = pl.program_id(0)

    @pl.when(i == 0)
    def _():
        acc_ref[...] = jnp.zeros_like(acc_ref)

    wdc = wdc_ref[...]
    d2 = jnp.dot(so_ref[...] * ncf_ref[...], qo_ref[...],
                 preferred_element_type=jnp.float32)
    density = jnp.concatenate([wdc[:, :NWAVE], d1_ref[...], d2], axis=1)
    h = _silu(jnp.dot(density, oW1_ref[...],
                      preferred_element_type=jnp.float32) + oB1_ref[...])
    atom = jnp.dot(h, oW2_ref[...], preferred_element_type=jnp.float32)
    aux = aux_ref[...]
    oh = _spec_onehot(aux)
    scv = jnp.dot(oh, scm_ref[...], preferred_element_type=jnp.float32)
    ae = (atom[:, 0:1] * scv[:, 0:1] + scv[:, 1:2]) * aux[:, 1:2]
    lane = lax.broadcasted_iota(jnp.int32, (1, G), 1).astype(jnp.float32)
    ohg = (aux[:, 2:3] == lane).astype(jnp.float32)
    acc_ref[...] += jnp.sum(ohg * ae, axis=0, keepdims=True)

    @pl.when(i == pl.num_programs(0) - 1)
    def _():
        out_ref[...] = acc_ref[...]


def _node_c(wdc_n, d1, ncf, sum_orb, aux, qones, oW1, oB1, oW2pad, scm):
    return pl.pallas_call(
        _nodeC_body,
        grid=(N // BN,),
        in_specs=[_neb(2 * NWAVE), _neb(NWAVE), _neb(PNORB * NWAVE),
                  _neb(PNORB * NWAVE), _neb(8), _nwb(qones.shape),
                  _nwb(oW1.shape), _nwb(oB1.shape), _nwb(oW2pad.shape),
                  _nwb(scm.shape)],
        out_specs=pl.BlockSpec((1, G), lambda i: (0, 0)),
        out_shape=jax.ShapeDtypeStruct((1, G), jnp.float32),
        scratch_shapes=[pltpu.VMEM((1, G), jnp.float32)],
    )(wdc_n, d1, ncf, sum_orb, aux, qones, oW1, oB1, oW2pad, scm)


def _expand_blocks(W, blocks):
    H = W.shape[0]
    Wb = W.reshape(H, -1, NWAVE)
    return Wb[:, list(blocks)].reshape(H, len(blocks) * NWAVE)


def kernel(cart, cell, disp_cell, neighlist, celllist, shiftimage,
           center_factor, species, params):
    p = params
    f32 = jnp.float32
    com_spec = jnp.array([[float(i), float(j)] for i in range(NSPEC)
                          for j in range(NSPEC)], dtype=f32)

    symm_cell = (disp_cell + jnp.transpose(disp_cell, (0, 2, 1))) / 2.0
    cell = cell + jnp.einsum('ijk,ikm->ijm', cell, symm_cell)
    symm_cell_n = symm_cell[celllist]
    cart = cart + jnp.einsum('ij,ijk->ik', cart, symm_cell_n)
    cellm = jnp.concatenate(
        [cell.reshape(G, 9), jnp.zeros((G, 7), f32)], axis=1)

    pad_idx = jnp.full((EP - E,), _PAD_NODE, jnp.int32)
    idx0 = jnp.concatenate([neighlist[0], pad_idx]).reshape(_NW, _NCHUNK, CH)
    idx1 = jnp.concatenate([neighlist[1], pad_idx]).reshape(_NW, _NCHUNK, CH)
    spec_idx = species

    node_tab = jnp.concatenate(
        [cart, jnp.zeros((N, 1), f32), spec_idx[:, None].astype(f32),
         celllist[:, None].astype(f32), jnp.zeros((N, 10), f32)], axis=1)
    gs, gd = _sc_gather2(node_tab, idx0, idx1)

    shp = jnp.concatenate(
        [shiftimage, jnp.zeros((3, EP - E), f32)], axis=1)
    shp = jnp.concatenate([shp, jnp.zeros((5, EP), f32)], axis=0)

    pair_spec = _silu(com_spec @ p['ncW1'] + p['ncB1']) @ p['ncW2'] + p['ncB2']
    embt = (_silu(pair_spec @ p['nnW1'] + p['nnB1']) @ p['nnW2']
            + p['nnB2'])
    ieadt = _silu(pair_spec @ p['rwW1']) @ p['rwW2']

    psA_np = np.zeros((NWAVE, 5 * NWAVE), np.float32)
    psB_np = np.zeros((NWAVE, 4 * NWAVE), np.float32)
    for j in range(5):
        psA_np[j, j * NWAVE:(j + 1) * NWAVE] = 1.0
    for j in range(5, PNORB):
        psB_np[j, (j - 5) * NWAVE:(j - 4) * NWAVE] = 1.0
    psA = jnp.asarray(psA_np)
    psB = jnp.asarray(psB_np)
    q_np = np.zeros((PNORB * NWAVE, NWAVE), np.float32)
    for j in range(PNORB):
        for m in range(NWAVE):
            q_np[j * NWAVE + m, m] = 1.0 / np.sqrt(2.0)
    qmat = jnp.asarray(q_np)

    rdW2 = p['rdW2']
    rdW2a = rdW2[:, 3 * NWAVE:5 * NWAVE]
    rdW2bA = _expand_blocks(rdW2, PIDX[:5])
    rdW2bB = _expand_blocks(rdW2, PIDX[5:])
    rxblocks = [r * PRMAXL + PIDX[j] for r in range(3) for j in range(PNORB)]
    e2W2e = _expand_blocks(p['ead2W2'], rxblocks)
    mpW2e = [_expand_blocks(p['mp0W2'], PIDX),
             _expand_blocks(p['mp1W2'], PIDX)]
    ead0W2e = _expand_blocks(p['ead0W2'], rxblocks)

    geo, ead0, wdc, worbA, worbB, rad = _edge_phase1(
        gs, gd, shp, cellm, embt, ieadt, p['rdW1'], rdW2a, rdW2bA, rdW2bB,
        p['ead2W1'], e2W2e, psA, psB)

    wdc_n = _sc_scatter_multi([wdc], idx0)[0]
    corb_raw = jnp.concatenate(
        [_sc_scatter_multi([worbA], idx0)[0],
         _sc_scatter_multi([worbB], idx0)[0]], axis=1)

    rt3 = np.sqrt(float(PRMAXL))
    eye9 = jnp.eye(PNORB, dtype=f32)
    bdA = jnp.stack(
        [jnp.kron(eye9, p['spec_coeff'][s_] / np.sqrt(float(NWAVE))) / rt3
         for s_ in range(NSPEC)])
    cc0 = p['contract_coeff'][0]
    bd0 = jnp.stack(
        [jnp.kron(eye9, cc0[s_, 0] / np.sqrt(float(NWAVE))) for s_ in
         range(NSPEC)])
    bd1 = jnp.stack(
        [jnp.kron(eye9, cc0[s_, 1] * rt3) for s_ in range(NSPEC)])
    ltab = (jnp.transpose(p['l_coeff'][0], (1, 0, 2))
            .reshape(NSPEC, PNORB * NWAVE) / rt3)
    qo_np = np.zeros((PNORB * NWAVE, NWAVE), np.float32)
    for j in range(PNORB):
        for m in range(NWAVE):
            qo_np[j * NWAVE + m, m] = 1.0
    qones = jnp.asarray(qo_np)
    aux = jnp.concatenate(
        [spec_idx[:, None].astype(f32), center_factor[:, None],
         celllist[:, None].astype(f32), jnp.zeros((N, 5), f32)], axis=1)
    scl = p['scale'].reshape(NSPEC, 2)
    scm = jnp.stack([scl[:, 0], p['outB2'][0] * scl[:, 0] + scl[:, 1]],
                    axis=1)
    oW2pad = jnp.concatenate(
        [p['outW2'], jnp.zeros((p['outW2'].shape[0], G - 1), f32)], axis=1)
    oB1 = p['outB1'][None, :]

    ncf = _node_a(wdc_n, corb_raw, aux, bdA)

    nc0, nc1 = _sc_gather2(ncf, idx0, idx1)
    res = _edge_phase2([ead0], geo, rad, nc0, nc1,
                       (p['mp0W1'], mpW2e[0]), psA, psB, qmat,
                       (p['ead0W1'], ead0W2e))
    ne1, orbA0, orbB0, nworbA, nworbB, rad = res
    sum_orb0 = jnp.concatenate(
        [_sc_scatter_multi([orbA0], idx0)[0],
         _sc_scatter_multi([orbB0], idx0)[0]], axis=1)
    sum_new = jnp.concatenate(
        [_sc_scatter_multi([nworbA], idx0)[0],
         _sc_scatter_multi([nworbB], idx0)[0]], axis=1)
    d1, ncf = _node_b(wdc_n, ncf, sum_orb0, sum_new, aux, bd0, bd1, ltab,
                      qones)

    nc0, nc1 = _sc_gather2(ncf, idx0, idx1)
    ne2, orbA1, orbB1 = _edge_phase2([ead0, ne1], geo, rad, nc0, nc1,
                                     None, psA, psB, qmat, None)
    sum_orb1 = jnp.concatenate(
        [_sc_scatter_multi([orbA1], idx0)[0],
         _sc_scatter_multi([orbB1], idx0)[0]], axis=1)

    energy = _node_c(wdc_n, d1, ncf, sum_orb1, aux, qones,
                     p['outW1'], oB1, oW2pad, scm)
    return energy.reshape(G)
